# Initial kernel scaffold; baseline (speedup 1.0000x reference)
import jax
import jax.numpy as jnp
from jax.experimental import pallas as pl


def kernel(x, edge_index, Wl0, bl0, Wr0, br0, att0, bias0, g0, be0, rm0, rv0, Wl1, bl1, Wr1, br1, att1, bias1, g1, be1, rm1, rv1, W1, b1, W2, b2):
    raise NotImplementedError("write your pallas kernel here")


# speedup vs baseline: 1.0850x; 1.0850x over previous
import functools

import jax
import jax.numpy as jnp
from jax.experimental import pallas as pl
from jax.experimental.pallas import tpu as pltpu

N = 10000
E = 160000
D = 256
H = 4
C = 256
HD = H * C
OUT = 128

NPAD = 10240
BM = 512


def _mm_kernel(x_ref, w_ref, b_ref, o_ref, *, act):
    acc = jnp.dot(x_ref[...], w_ref[...], preferred_element_type=jnp.float32)
    acc = acc + b_ref[...][None, :]
    if act == "relu":
        acc = jnp.maximum(acc, 0.0)
    o_ref[...] = acc


def _mm(x, w, b, act=None):
    M, K = x.shape
    Nout = w.shape[1]
    return pl.pallas_call(
        functools.partial(_mm_kernel, act=act),
        grid=(M // BM,),
        in_specs=[
            pl.BlockSpec((BM, K), lambda i: (i, 0)),
            pl.BlockSpec((K, Nout), lambda i: (0, 0)),
            pl.BlockSpec((Nout,), lambda i: (0,)),
        ],
        out_specs=pl.BlockSpec((BM, Nout), lambda i: (i, 0)),
        out_shape=jax.ShapeDtypeStruct((M, Nout), jnp.float32),
    )(x, w, b)


def _gat_edges(xl, xr, src, dst, att, bias):
    xl4 = xl.reshape(N, H, C)
    xr4 = xr.reshape(N, H, C)
    e = jax.nn.leaky_relu(xl4[src] + xr4[dst], 0.2)
    logits = jnp.sum(e * att[None, :, :], axis=-1)
    p = jnp.exp(logits)
    denom = jax.ops.segment_sum(p, dst, num_segments=N)
    acc = jax.ops.segment_sum(xl4[src] * p[:, :, None], dst, num_segments=N)
    out = acc / (denom[:, :, None] + 1e-16)
    return out.reshape(N, HD) + bias


def kernel(x, edge_index, Wl0, bl0, Wr0, br0, att0, bias0, g0, be0, rm0, rv0,
           Wl1, bl1, Wr1, br1, att1, bias1, g1, be1, rm1, rv1, W1, b1, W2, b2):
    loop = jnp.arange(N, dtype=edge_index.dtype)
    src = jnp.concatenate([edge_index[0], loop])
    dst = jnp.concatenate([edge_index[1], loop])

    s0 = g0 * jax.lax.rsqrt(rv0 + 1e-5)
    t0 = be0 - rm0 * s0
    s1 = g1 * jax.lax.rsqrt(rv1 + 1e-5)
    t1 = be1 - rm1 * s1

    xp = jnp.pad(x, ((0, NPAD - N), (0, 0)))
    W0 = jnp.concatenate([Wl0, Wr0], axis=1)
    bb0 = jnp.concatenate([bl0, br0])
    lr0 = _mm(xp, W0, bb0)[:N]
    h = _gat_edges(lr0[:, :HD], lr0[:, HD:], src, dst, att0, bias0)
    h = jnp.maximum(h * s0 + t0, 0.0)

    hp = jnp.pad(h, ((0, NPAD - N), (0, 0)))
    W1c = jnp.concatenate([Wl1, Wr1], axis=1)
    bb1 = jnp.concatenate([bl1, br1])
    lr1 = _mm(hp, W1c, bb1)[:N]
    h = _gat_edges(lr1[:, :HD], lr1[:, HD:], src, dst, att1, bias1)
    h = jnp.maximum(h * s1 + t1, 0.0)

    hp = jnp.pad(h, ((0, NPAD - N), (0, 0)))
    h = _mm(hp, W1, b1, act="relu")
    h = _mm(h, W2, b2)[:N]
    return h


# speedup vs baseline: 4.7472x; 4.3752x over previous
import functools

import jax
import jax.numpy as jnp
from jax import lax
from jax.experimental import pallas as pl
from jax.experimental.pallas import tpu as pltpu
from jax.experimental.pallas import tpu_sc as plsc

i32 = jnp.int32
f32 = jnp.float32

N = 10000
E = 160000
D = 256
H = 4
C = 256
HD = H * C
OUT = 128

NW = 32
S = 5344
EPT = NW * S
S2 = 10240
BINS = 320
BSH = 5
BNB = 32
PK = 16384
NPAD = BINS * BNB
AW = HD + 16
BPT = BINS // NW
BM = 512


def _mm_kernel(x_ref, w_ref, b_ref, o_ref, *, act):
    acc = jnp.dot(x_ref[...], w_ref[...], preferred_element_type=f32)
    acc = acc + b_ref[...][None, :]
    if act == "relu":
        acc = jnp.maximum(acc, 0.0)
    o_ref[...] = acc


def _mm(x, w, b, act=None):
    M, K = x.shape
    Nout = w.shape[1]
    return pl.pallas_call(
        functools.partial(_mm_kernel, act=act),
        grid=(M // BM,),
        in_specs=[
            pl.BlockSpec((BM, K), lambda i: (i, 0)),
            pl.BlockSpec((K, Nout), lambda i: (0, 0)),
            pl.BlockSpec((Nout,), lambda i: (0,)),
        ],
        out_specs=pl.BlockSpec((BM, Nout), lambda i: (i, 0)),
        out_shape=jax.ShapeDtypeStruct((M, Nout), f32),
    )(x, w, b)


def _mm2_kernel(x_ref, wl_ref, bl_ref, wr_ref, br_ref, ol_ref, or_ref):
    xv = x_ref[...]
    ol_ref[...] = jnp.dot(xv, wl_ref[...], preferred_element_type=f32) + bl_ref[...][None, :]
    or_ref[...] = jnp.dot(xv, wr_ref[...], preferred_element_type=f32) + br_ref[...][None, :]


def _mm2(x, wl, bl, wr, br):
    M, K = x.shape
    return pl.pallas_call(
        _mm2_kernel,
        grid=(M // BM,),
        in_specs=[
            pl.BlockSpec((BM, K), lambda i: (i, 0)),
            pl.BlockSpec((K, HD), lambda i: (0, 0)),
            pl.BlockSpec((HD,), lambda i: (0,)),
            pl.BlockSpec((K, HD), lambda i: (0, 0)),
            pl.BlockSpec((HD,), lambda i: (0,)),
        ],
        out_specs=[
            pl.BlockSpec((BM, HD), lambda i: (i, 0)),
            pl.BlockSpec((BM, HD), lambda i: (i, 0)),
        ],
        out_shape=[
            jax.ShapeDtypeStruct((M, HD), f32),
            jax.ShapeDtypeStruct((M, HD), f32),
        ],
    )(x, wl, bl, wr, br)


def _bucket_body(epk, bpack, bstarts, bends, spk, stage, zi, hist, sem):
    c = lax.axis_index("c")
    s = lax.axis_index("s")
    w = c * 16 + s
    iota = lax.iota(i32, 16)
    pltpu.sync_copy(epk.at[pl.ds(pl.multiple_of(w * S, 32), S)], spk)

    for r in range(BINS):
        hist[r] = jnp.int32(0)

    def count_body(k, cy):
        bv = (spk[pl.ds(k * 16, 16)] & (PK - 1)) >> BSH
        for q in range(16):
            b = bv[q]
            hist[b] = hist[b] + 1
        return cy

    lax.fori_loop(0, S // 16, count_body, 0)

    off = jnp.int32(0)
    starts = []
    ends = []
    for r in range(BINS):
        cnt = hist[r]
        starts.append(off)
        ends.append(off + cnt)
        hist[r] = off
        off = off + ((cnt + 15) >> 4 << 4)

    for g in range(BINS // 16):
        sv = jnp.zeros((16,), i32)
        evv = jnp.zeros((16,), i32)
        for q in range(16):
            sv = jnp.where(iota == q, jnp.full((16,), starts[g * 16 + q], i32), sv)
            evv = jnp.where(iota == q, jnp.full((16,), ends[g * 16 + q], i32), evv)
        stage[pl.ds(0, 16)] = sv
        stage[pl.ds(16, 16)] = evv
        oidx = (g * 16 + iota) * NW + w
        pltpu.sync_copy(stage.at[pl.ds(0, 16)], bstarts.at[oidx])
        pltpu.sync_copy(stage.at[pl.ds(16, 16)], bends.at[oidx])

    gbase = w * S2

    def scat_body(k, cy):
        bv = (spk[pl.ds(k * 16, 16)] & (PK - 1)) >> BSH
        posv = jnp.zeros((16,), i32)
        for q in range(16):
            b = bv[q]
            p = hist[b]
            hist[b] = p + 1
            posv = jnp.where(iota == q, jnp.full((16,), p, i32), posv)
        pltpu.sync_copy(spk.at[pl.ds(k * 16, 16)], bpack.at[gbase + posv])
        return cy

    lax.fori_loop(0, S // 16, scat_body, 0)

    @pl.when(w == 0)
    def _tail():
        zi[pl.ds(0, 16)] = jnp.zeros((16,), i32)
        pltpu.sync_copy(zi, bpack.at[pl.ds(NW * S2, 16)])


def _make_bucket_kernel():
    mesh = plsc.VectorSubcoreMesh(core_axis_name="c", subcore_axis_name="s")
    return pl.kernel(
        _bucket_body,
        out_type=(
            jax.ShapeDtypeStruct((NW * S2 + 16,), i32),
            jax.ShapeDtypeStruct((BINS * NW,), i32),
            jax.ShapeDtypeStruct((BINS * NW,), i32),
        ),
        mesh=mesh,
        scratch_types=[
            pltpu.VMEM((S,), i32),
            pltpu.VMEM((32,), i32),
            pltpu.VMEM((16,), i32),
            pltpu.SMEM((BINS,), i32),
            pltpu.SemaphoreType.DMA,
        ],
    )


def _layer_body(xl, xr, bpack, bstarts, bends, atth, pah, pbh, hout,
                attv, pav, pbv, xlr, xrr, acc, pkb, rvb, stv, env,
                smst, smen, sem):
    c = lax.axis_index("c")
    s = lax.axis_index("s")
    tid = c * 16 + s
    iota = lax.iota(i32, 16)
    zf = jnp.zeros((16,), f32)

    pltpu.sync_copy(atth, attv)
    pltpu.sync_copy(pah, pav)
    pltpu.sync_copy(pbh, pbv)

    def bin_body(bi, carry):
        bn = tid * BPT + bi
        base = bn * BNB

        bno = pl.multiple_of(bn * NW, 32)
        pltpu.sync_copy(bstarts.at[pl.ds(bno, NW)], stv)
        pltpu.sync_copy(bends.at[pl.ds(bno, NW)], env)
        sv0 = stv[pl.ds(0, 16)]
        sv1 = stv[pl.ds(16, 16)]
        ev0 = env[pl.ds(0, 16)]
        ev1 = env[pl.ds(16, 16)]
        for q in range(16):
            smst[q] = sv0[q]
            smst[16 + q] = sv1[q]
            smen[q] = ev0[q]
            smen[16 + q] = ev1[q]

        def zb(row, cy):
            for g in range(AW // 16):
                acc[row, pl.ds(g * 16, 16)] = zf
            return cy
        lax.fori_loop(0, BNB, zb, 0)

        def w_body(w, cy):
            st = smst[w]
            en = smen[w]
            nch = (en - st + 15) // 16

            def chunk_body(k, cy2):
                pos = pl.multiple_of(w * S2 + st + k * 16, 16)
                pltpu.sync_copy(bpack.at[pl.ds(pos, 16)], pkb)
                pkv = pkb[pl.ds(0, 16)]
                srcv = jnp.clip(pkv >> 14, 0, NPAD - 1)
                dstv = jnp.minimum(pkv & (PK - 1), NPAD - 1)
                d1 = pltpu.async_copy(xl.at[srcv], xlr, sem)
                d2 = pltpu.async_copy(xr.at[dstv], xrr, sem)
                d1.wait()
                d2.wait()
                rvb[pl.ds(0, 16)] = dstv - base
                nv = jnp.minimum(en - st - k * 16, 16)

                def edge_body(e, cy3):
                    row = rvb[pl.ds(e, 16)][0]
                    pvs = []
                    for h in range(H):
                        part = zf
                        for j in range(C // 16):
                            col = h * C + j * 16
                            t = xlr[e, pl.ds(col, 16)] + xrr[e, pl.ds(col, 16)]
                            t = jnp.where(t > 0, t, 0.2 * t)
                            part = part + t * attv[pl.ds(col, 16)]
                        tree = [part[q] for q in range(16)]
                        while len(tree) > 1:
                            tree = [tree[2 * q] + tree[2 * q + 1]
                                    for q in range(len(tree) // 2)]
                        pv = jnp.exp(jnp.full((16,), tree[0], f32))
                        pvs.append(pv)
                        for j in range(C // 16):
                            col = h * C + j * 16
                            acc[row, pl.ds(col, 16)] = acc[row, pl.ds(col, 16)] \
                                + pv * xlr[e, pl.ds(col, 16)]
                    ptail = jnp.where(iota == 0, pvs[0],
                             jnp.where(iota == 1, pvs[1],
                              jnp.where(iota == 2, pvs[2],
                               jnp.where(iota == 3, pvs[3], zf))))
                    acc[row, pl.ds(HD, 16)] = acc[row, pl.ds(HD, 16)] + ptail
                    return cy3

                lax.fori_loop(0, nv, edge_body, 0)
                return cy2

            lax.fori_loop(0, nch, chunk_body, 0)
            return cy

        lax.fori_loop(0, NW, w_body, 0)

        def nr(row, cy):
            dinvv = 1.0 / (acc[row, pl.ds(HD, 16)] + 1e-16)
            for h in range(H):
                dv = jnp.full((16,), dinvv[h], f32)
                for j in range(C // 16):
                    col = h * C + j * 16
                    o = acc[row, pl.ds(col, 16)] * dv * pav[pl.ds(col, 16)] \
                        + pbv[pl.ds(col, 16)]
                    acc[row, pl.ds(col, 16)] = jnp.maximum(o, 0.0)
            return cy
        lax.fori_loop(0, BNB, nr, 0)

        nrows = jnp.clip(N - base, 0, BNB)
        nchw = nrows // 16

        def wr(q, cy):
            pltpu.sync_copy(acc.at[pl.ds(q * 16, 16)],
                            hout.at[pl.ds(base + q * 16, 16)])
            return cy
        lax.fori_loop(0, nchw, wr, 0)
        return carry

    lax.fori_loop(0, BPT, bin_body, 0)


def _make_layer_kernel():
    mesh = plsc.VectorSubcoreMesh(core_axis_name="c", subcore_axis_name="s")
    return pl.kernel(
        _layer_body,
        out_type=jax.ShapeDtypeStruct((NPAD, AW), f32),
        mesh=mesh,
        scratch_types=[
            pltpu.VMEM((HD,), f32),
            pltpu.VMEM((HD,), f32),
            pltpu.VMEM((HD,), f32),
            pltpu.VMEM((16, HD), f32),
            pltpu.VMEM((16, HD), f32),
            pltpu.VMEM((BNB, AW), f32),
            pltpu.VMEM((16,), i32),
            pltpu.VMEM((32,), i32),
            pltpu.VMEM((NW,), i32),
            pltpu.VMEM((NW,), i32),
            pltpu.SMEM((NW,), i32),
            pltpu.SMEM((NW,), i32),
            pltpu.SemaphoreType.DMA,
        ],
    )


def kernel(x, edge_index, Wl0, bl0, Wr0, br0, att0, bias0, g0, be0, rm0, rv0,
           Wl1, bl1, Wr1, br1, att1, bias1, g1, be1, rm1, rv1, W1, b1, W2, b2):
    loop = jnp.arange(N, dtype=edge_index.dtype)
    srcp = jnp.concatenate([edge_index[0], loop,
                            jnp.zeros((EPT - E - N,), i32)])
    dstp = jnp.concatenate([edge_index[1], loop,
                            jnp.full((EPT - E - N,), NPAD - 1, i32)])
    epk = srcp * PK + dstp

    bucket = _make_bucket_kernel()
    bpack, bstarts, bends = bucket(epk)

    s0 = g0 * lax.rsqrt(rv0 + 1e-5)
    pb0 = bias0 * s0 + (be0 - rm0 * s0)
    s1 = g1 * lax.rsqrt(rv1 + 1e-5)
    pb1 = bias1 * s1 + (be1 - rm1 * s1)

    layer = _make_layer_kernel()

    xp = jnp.pad(x, ((0, NPAD - N), (0, 0)))
    xl0, xr0 = _mm2(xp, Wl0, bl0, Wr0, br0)
    h = layer(xl0, xr0, bpack, bstarts, bends, att0.reshape(HD), s0, pb0)

    hp = jnp.pad(h[:N, :HD], ((0, NPAD - N), (0, 0)))
    xl1, xr1 = _mm2(hp, Wl1, bl1, Wr1, br1)
    h = layer(xl1, xr1, bpack, bstarts, bends, att1.reshape(HD), s1, pb1)

    hp = jnp.pad(h[:N, :HD], ((0, NPAD - N), (0, 0)))
    h = _mm(hp, W1, b1, act="relu")
    h = _mm(h, W2, b2)[:N]
    return h


# speedup vs baseline: 6.1034x; 1.2857x over previous
import functools

import jax
import jax.numpy as jnp
from jax import lax
from jax.experimental import pallas as pl
from jax.experimental.pallas import tpu as pltpu
from jax.experimental.pallas import tpu_sc as plsc

i32 = jnp.int32
f32 = jnp.float32

N = 10000
E = 160000
D = 256
H = 4
C = 256
HD = H * C
OUT = 128

NW = 32
S = 5344
EPT = NW * S
BINS = 320
BSH = 5
BNB = 32
PK = 16384
NPAD = BINS * BNB
AW = HD + 16
BPT = BINS // NW
BPK = EPT + BINS * 16 + 512
SBE = 4096
BM = 512


def _mm_kernel(x_ref, w_ref, b_ref, o_ref, *, act):
    acc = jnp.dot(x_ref[...], w_ref[...], preferred_element_type=f32)
    acc = acc + b_ref[...][None, :]
    if act == "relu":
        acc = jnp.maximum(acc, 0.0)
    o_ref[...] = acc


def _mm(x, w, b, act=None):
    M, K = x.shape
    Nout = w.shape[1]
    return pl.pallas_call(
        functools.partial(_mm_kernel, act=act),
        grid=(M // BM,),
        in_specs=[
            pl.BlockSpec((BM, K), lambda i: (i, 0)),
            pl.BlockSpec((K, Nout), lambda i: (0, 0)),
            pl.BlockSpec((Nout,), lambda i: (0,)),
        ],
        out_specs=pl.BlockSpec((BM, Nout), lambda i: (i, 0)),
        out_shape=jax.ShapeDtypeStruct((M, Nout), f32),
    )(x, w, b)


def _mm2_kernel(x_ref, wl_ref, bl_ref, wr_ref, br_ref, ol_ref, or_ref):
    xv = x_ref[...]
    ol_ref[...] = jnp.dot(xv, wl_ref[...], preferred_element_type=f32) + bl_ref[...][None, :]
    or_ref[...] = jnp.dot(xv, wr_ref[...], preferred_element_type=f32) + br_ref[...][None, :]


def _mm2(x, wl, bl, wr, br):
    M, K = x.shape
    return pl.pallas_call(
        _mm2_kernel,
        grid=(M // BM,),
        in_specs=[
            pl.BlockSpec((BM, K), lambda i: (i, 0)),
            pl.BlockSpec((K, HD), lambda i: (0, 0)),
            pl.BlockSpec((HD,), lambda i: (0,)),
            pl.BlockSpec((K, HD), lambda i: (0, 0)),
            pl.BlockSpec((HD,), lambda i: (0,)),
        ],
        out_specs=[
            pl.BlockSpec((BM, HD), lambda i: (i, 0)),
            pl.BlockSpec((BM, HD), lambda i: (i, 0)),
        ],
        out_shape=[
            jax.ShapeDtypeStruct((M, HD), f32),
            jax.ShapeDtypeStruct((M, HD), f32),
        ],
    )(x, wl, bl, wr, br)


def _count_body(epk, cnts, spk, stage, hist, sem):
    c = lax.axis_index("c")
    s = lax.axis_index("s")
    w = c * 16 + s
    iota = lax.iota(i32, 16)
    pltpu.sync_copy(epk.at[pl.ds(pl.multiple_of(w * S, 32), S)], spk)

    for r in range(BINS):
        hist[r] = jnp.int32(0)

    def count_loop(k, cy):
        bv = (spk[pl.ds(k * 16, 16)] & (PK - 1)) >> BSH
        for q in range(16):
            b = bv[q]
            hist[b] = hist[b] + 1
        return cy

    lax.fori_loop(0, S // 16, count_loop, 0)

    def emit(g, cy):
        vv = jnp.zeros((16,), i32)
        for q in range(16):
            vv = jnp.where(iota == q, jnp.full((16,), hist[g * 16 + q], i32), vv)
        stage[pl.ds(0, 16)] = vv
        oidx = (g * 16 + iota) * NW + w
        pltpu.sync_copy(stage.at[pl.ds(0, 16)], cnts.at[oidx])
        return cy

    lax.fori_loop(0, BINS // 16, emit, 0)


def _make_count_kernel():
    mesh = plsc.VectorSubcoreMesh(core_axis_name="c", subcore_axis_name="s")
    return pl.kernel(
        _count_body,
        out_type=jax.ShapeDtypeStruct((BINS * NW,), i32),
        mesh=mesh,
        scratch_types=[
            pltpu.VMEM((S,), i32),
            pltpu.VMEM((16,), i32),
            pltpu.SMEM((BINS,), i32),
            pltpu.SemaphoreType.DMA,
        ],
    )


def _scatter_body(epk, cnts, bpack, bstarts, bends, spk, cbuf, stage, zi,
                  hist, sem):
    c = lax.axis_index("c")
    s = lax.axis_index("s")
    w = c * 16 + s
    iota = lax.iota(i32, 16)
    pltpu.sync_copy(epk.at[pl.ds(pl.multiple_of(w * S, 32), S)], spk)
    pltpu.sync_copy(cnts, cbuf)

    def scan(b, gstart):
        v0 = cbuf[pl.ds(pl.multiple_of(b * NW, 32), 16)]
        v1 = cbuf[pl.ds(pl.multiple_of(b * NW, 32) + 16, 16)]
        tot = jnp.int32(0)
        woff = jnp.int32(0)
        for q in range(16):
            cq = v0[q]
            woff = woff + jnp.where(jnp.int32(q) < w, cq, 0)
            tot = tot + cq
        for q in range(16):
            cq = v1[q]
            woff = woff + jnp.where(jnp.int32(16 + q) < w, cq, 0)
            tot = tot + cq
        hist[b] = gstart + woff
        hist[BINS + b] = gstart + tot
        return gstart + ((tot + 15) >> 4 << 4)

    lax.fori_loop(0, BINS, scan, jnp.int32(0))

    @pl.when(w == 0)
    def _emit_bounds():
        def emit(g, cy):
            sv = jnp.zeros((16,), i32)
            evv = jnp.zeros((16,), i32)
            for q in range(16):
                st_q = hist[g * 16 + q]
                en_q = hist[BINS + g * 16 + q]
                sv = jnp.where(iota == q, jnp.full((16,), st_q, i32), sv)
                evv = jnp.where(iota == q, jnp.full((16,), en_q, i32), evv)
            stage[pl.ds(0, 16)] = sv
            stage[pl.ds(16, 16)] = evv
            pltpu.sync_copy(stage.at[pl.ds(0, 16)],
                            bstarts.at[pl.ds(pl.multiple_of(g * 16, 16), 16)])
            pltpu.sync_copy(stage.at[pl.ds(16, 16)],
                            bends.at[pl.ds(pl.multiple_of(g * 16, 16), 16)])
            return cy
        lax.fori_loop(0, BINS // 16, emit, 0)

    def scat_body(k, cy):
        bv = (spk[pl.ds(k * 16, 16)] & (PK - 1)) >> BSH
        posv = jnp.zeros((16,), i32)
        for q in range(16):
            b = bv[q]
            p = hist[b]
            hist[b] = p + 1
            posv = jnp.where(iota == q, jnp.full((16,), p, i32), posv)
        pltpu.sync_copy(spk.at[pl.ds(k * 16, 16)], bpack.at[posv])
        return cy

    lax.fori_loop(0, S // 16, scat_body, 0)


def _make_scatter_kernel():
    mesh = plsc.VectorSubcoreMesh(core_axis_name="c", subcore_axis_name="s")
    return pl.kernel(
        _scatter_body,
        out_type=(
            jax.ShapeDtypeStruct((BPK,), i32),
            jax.ShapeDtypeStruct((BINS,), i32),
            jax.ShapeDtypeStruct((BINS,), i32),
        ),
        mesh=mesh,
        scratch_types=[
            pltpu.VMEM((S,), i32),
            pltpu.VMEM((BINS * NW,), i32),
            pltpu.VMEM((32,), i32),
            pltpu.VMEM((16,), i32),
            pltpu.SMEM((2 * BINS,), i32),
            pltpu.SemaphoreType.DMA,
        ],
    )


def _layer_body(xl, xr, bpack, bstarts, bends, atth, pah, pbh, hout,
                attv, pav, pbv, xlr0, xrr0, xlr1, xrr1, acc, ebuf, rvb,
                stv, smb, sem0, sem1):
    c = lax.axis_index("c")
    s = lax.axis_index("s")
    tid = c * 16 + s
    iota = lax.iota(i32, 16)
    zf = jnp.zeros((16,), f32)

    pltpu.sync_copy(atth, attv)
    pltpu.sync_copy(pah, pav)
    pltpu.sync_copy(pbh, pbv)

    b0 = tid * BPT
    off8 = pl.multiple_of((b0 >> 3) << 3, 8)
    shift = b0 - off8
    pltpu.sync_copy(bstarts.at[pl.ds(off8, 16)], stv.at[pl.ds(0, 16)])
    pltpu.sync_copy(bends.at[pl.ds(off8, 16)], stv.at[pl.ds(16, 16)])
    sv = stv[pl.ds(0, 16)]
    evv = stv[pl.ds(16, 16)]
    for q in range(16):
        smb[q] = sv[q]
        smb[16 + q] = evv[q]

    def issue(pos16, bxl, bxr, semx):
        pkv = ebuf[pl.ds(pos16, 16)]
        srcv = jnp.clip(pkv >> 14, 0, NPAD - 1)
        dstv = jnp.minimum(pkv & (PK - 1), NPAD - 1)
        pltpu.async_copy(xl.at[srcv], bxl, semx)
        pltpu.async_copy(xr.at[dstv], bxr, semx)

    def wait(bxl, bxr, semx):
        pltpu.make_async_copy(xl.at[iota], bxl, semx).wait()
        pltpu.make_async_copy(xr.at[iota], bxr, semx).wait()

    def bin_body(bi, carry):
        bn = tid * BPT + bi
        base = bn * BNB
        st = smb[shift + bi]
        en = smb[16 + shift + bi]
        cnt = en - st

        def zb(row, cy):
            for g in range(AW // 16):
                acc[row, pl.ds(g * 16, 16)] = zf
            return cy
        lax.fori_loop(0, BNB, zb, 0)

        def compute(cidx, local16, bxl, bxr):
            pkv = ebuf[pl.ds(local16, 16)]
            dstv = jnp.minimum(pkv & (PK - 1), NPAD - 1)
            rvb[pl.ds(0, 16)] = dstv - base
            nv = jnp.clip(cnt - cidx * 16, 0, 16)

            def edge_body(e, cy3):
                row = rvb[pl.ds(e, 16)][0]
                pvs = []
                for h in range(H):
                    part = zf
                    for j in range(C // 16):
                        col = h * C + j * 16
                        t = bxl[e, pl.ds(col, 16)] + bxr[e, pl.ds(col, 16)]
                        t = jnp.where(t > 0, t, 0.2 * t)
                        part = part + t * attv[pl.ds(col, 16)]
                    tree = [part[q] for q in range(16)]
                    while len(tree) > 1:
                        tree = [tree[2 * q] + tree[2 * q + 1]
                                for q in range(len(tree) // 2)]
                    pv = jnp.exp(jnp.full((16,), tree[0], f32))
                    pvs.append(pv)
                    for j in range(C // 16):
                        col = h * C + j * 16
                        acc[row, pl.ds(col, 16)] = acc[row, pl.ds(col, 16)] \
                            + pv * bxl[e, pl.ds(col, 16)]
                ptail = jnp.where(iota == 0, pvs[0],
                         jnp.where(iota == 1, pvs[1],
                          jnp.where(iota == 2, pvs[2],
                           jnp.where(iota == 3, pvs[3], zf))))
                acc[row, pl.ds(HD, 16)] = acc[row, pl.ds(HD, 16)] + ptail
                return cy3

            lax.fori_loop(0, nv, edge_body, 0)

        nsb = (cnt + SBE - 1) // SBE

        def sb_body(sb, cy):
            sbase = pl.multiple_of(st + sb * SBE, 16)
            rem = cnt - sb * SBE
            nblk = jnp.minimum((rem + 255) // 256, SBE // 256)

            def ld(t2, cy2):
                o = pl.multiple_of(t2 * 256, 256)
                pltpu.sync_copy(bpack.at[pl.ds(sbase + o, 256)],
                                ebuf.at[pl.ds(o, 256)])
                return cy2
            lax.fori_loop(0, nblk, ld, 0)

            nchk = jnp.clip((rem + 15) // 16, 0, SBE // 16)

            @pl.when(nchk > 0)
            def _pro():
                issue(0, xlr0, xrr0, sem0)

            def pair_body(m, cy2):
                c0 = 2 * m
                c1 = 2 * m + 1
                cond1 = c1 < nchk
                cond2 = c1 + 1 < nchk
                wait(xlr0, xrr0, sem0)

                @pl.when(cond1)
                def _i1():
                    issue(c1 * 16, xlr1, xrr1, sem1)

                compute(sb * (SBE // 16) + c0, c0 * 16, xlr0, xrr0)

                @pl.when(cond2)
                def _i2():
                    issue((c1 + 1) * 16, xlr0, xrr0, sem0)

                @pl.when(cond1)
                def _c1():
                    wait(xlr1, xrr1, sem1)
                    compute(sb * (SBE // 16) + c1, c1 * 16, xlr1, xrr1)

                return cy2

            lax.fori_loop(0, (nchk + 1) // 2, pair_body, 0)
            return cy

        lax.fori_loop(0, nsb, sb_body, 0)

        def nr(row, cy):
            dinvv = 1.0 / (acc[row, pl.ds(HD, 16)] + 1e-16)
            for h in range(H):
                dv = jnp.full((16,), dinvv[h], f32)
                for j in range(C // 16):
                    col = h * C + j * 16
                    o = acc[row, pl.ds(col, 16)] * dv * pav[pl.ds(col, 16)] \
                        + pbv[pl.ds(col, 16)]
                    acc[row, pl.ds(col, 16)] = jnp.maximum(o, 0.0)
            return cy
        lax.fori_loop(0, BNB, nr, 0)

        nrows = jnp.clip(N - base, 0, BNB)
        nchw = nrows // 16

        def wr(q, cy):
            pltpu.sync_copy(acc.at[pl.ds(q * 16, 16)],
                            hout.at[pl.ds(base + q * 16, 16)])
            return cy
        lax.fori_loop(0, nchw, wr, 0)
        return carry

    lax.fori_loop(0, BPT, bin_body, 0)


def _make_layer_kernel():
    mesh = plsc.VectorSubcoreMesh(core_axis_name="c", subcore_axis_name="s")
    return pl.kernel(
        _layer_body,
        out_type=jax.ShapeDtypeStruct((NPAD, AW), f32),
        mesh=mesh,
        scratch_types=[
            pltpu.VMEM((HD,), f32),
            pltpu.VMEM((HD,), f32),
            pltpu.VMEM((HD,), f32),
            pltpu.VMEM((16, HD), f32),
            pltpu.VMEM((16, HD), f32),
            pltpu.VMEM((16, HD), f32),
            pltpu.VMEM((16, HD), f32),
            pltpu.VMEM((BNB, AW), f32),
            pltpu.VMEM((SBE,), i32),
            pltpu.VMEM((32,), i32),
            pltpu.VMEM((32,), i32),
            pltpu.SMEM((32,), i32),
            pltpu.SemaphoreType.DMA,
            pltpu.SemaphoreType.DMA,
        ],
    )


def kernel(x, edge_index, Wl0, bl0, Wr0, br0, att0, bias0, g0, be0, rm0, rv0,
           Wl1, bl1, Wr1, br1, att1, bias1, g1, be1, rm1, rv1, W1, b1, W2, b2):
    loop = jnp.arange(N, dtype=edge_index.dtype)
    srcp = jnp.concatenate([edge_index[0], loop,
                            jnp.zeros((EPT - E - N,), i32)])
    dstp = jnp.concatenate([edge_index[1], loop,
                            jnp.full((EPT - E - N,), NPAD - 1, i32)])
    epk = srcp * PK + dstp

    cnts = _make_count_kernel()(epk)
    bpack, bstarts, bends = _make_scatter_kernel()(epk, cnts)

    s0 = g0 * lax.rsqrt(rv0 + 1e-5)
    pb0 = bias0 * s0 + (be0 - rm0 * s0)
    s1 = g1 * lax.rsqrt(rv1 + 1e-5)
    pb1 = bias1 * s1 + (be1 - rm1 * s1)

    layer = _make_layer_kernel()

    xp = jnp.pad(x, ((0, NPAD - N), (0, 0)))
    xl0, xr0 = _mm2(xp, Wl0, bl0, Wr0, br0)
    h = layer(xl0, xr0, bpack, bstarts, bends, att0.reshape(HD), s0, pb0)

    hp = jnp.pad(h[:N, :HD], ((0, NPAD - N), (0, 0)))
    xl1, xr1 = _mm2(hp, Wl1, bl1, Wr1, br1)
    h = layer(xl1, xr1, bpack, bstarts, bends, att1.reshape(HD), s1, pb1)

    hp = jnp.pad(h[:N, :HD], ((0, NPAD - N), (0, 0)))
    h = _mm(hp, W1, b1, act="relu")
    h = _mm(h, W2, b2)[:N]
    return h


# speedup vs baseline: 6.9313x; 1.1357x over previous
import functools

import jax
import jax.numpy as jnp
from jax import lax
from jax.experimental import pallas as pl
from jax.experimental.pallas import tpu as pltpu
from jax.experimental.pallas import tpu_sc as plsc

i32 = jnp.int32
f32 = jnp.float32

N = 10000
E = 160000
D = 256
H = 4
C = 256
HD = H * C
OUT = 128

NW = 32
S = 5344
EPT = NW * S
BINS = 320
BSH = 5
BNB = 32
PK = 16384
NPAD = BINS * BNB
AW = HD + 16
BPT = BINS // NW
BPK = EPT + BINS * 16 + 512
SBE = 4096
BM = 512


def _mm_kernel(x_ref, w_ref, b_ref, o_ref, *, act):
    acc = jnp.dot(x_ref[...], w_ref[...], preferred_element_type=f32)
    acc = acc + b_ref[...][None, :]
    if act == "relu":
        acc = jnp.maximum(acc, 0.0)
    o_ref[...] = acc


def _mm(x, w, b, act=None):
    M, K = x.shape
    Nout = w.shape[1]
    return pl.pallas_call(
        functools.partial(_mm_kernel, act=act),
        grid=(M // BM,),
        in_specs=[
            pl.BlockSpec((BM, K), lambda i: (i, 0)),
            pl.BlockSpec((K, Nout), lambda i: (0, 0)),
            pl.BlockSpec((Nout,), lambda i: (0,)),
        ],
        out_specs=pl.BlockSpec((BM, Nout), lambda i: (i, 0)),
        out_shape=jax.ShapeDtypeStruct((M, Nout), f32),
    )(x, w, b)


def _mm2_kernel(x_ref, wl_ref, bl_ref, wr_ref, br_ref, ol_ref, or_ref):
    xv = x_ref[...]
    ol_ref[...] = jnp.dot(xv, wl_ref[...], preferred_element_type=f32) + bl_ref[...][None, :]
    or_ref[...] = jnp.dot(xv, wr_ref[...], preferred_element_type=f32) + br_ref[...][None, :]


def _mm2(x, wl, bl, wr, br):
    M, K = x.shape
    return pl.pallas_call(
        _mm2_kernel,
        grid=(M // BM,),
        in_specs=[
            pl.BlockSpec((BM, K), lambda i: (i, 0)),
            pl.BlockSpec((K, HD), lambda i: (0, 0)),
            pl.BlockSpec((HD,), lambda i: (0,)),
            pl.BlockSpec((K, HD), lambda i: (0, 0)),
            pl.BlockSpec((HD,), lambda i: (0,)),
        ],
        out_specs=[
            pl.BlockSpec((BM, HD), lambda i: (i, 0)),
            pl.BlockSpec((BM, HD), lambda i: (i, 0)),
        ],
        out_shape=[
            jax.ShapeDtypeStruct((M, HD), f32),
            jax.ShapeDtypeStruct((M, HD), f32),
        ],
    )(x, wl, bl, wr, br)


def _count_body(epk, cnts, spk, stage, hist, sem):
    c = lax.axis_index("c")
    s = lax.axis_index("s")
    w = c * 16 + s
    iota = lax.iota(i32, 16)
    pltpu.sync_copy(epk.at[pl.ds(pl.multiple_of(w * S, 32), S)], spk)

    for r in range(BINS):
        hist[r] = jnp.int32(0)

    def count_loop(k, cy):
        bv = (spk[pl.ds(k * 16, 16)] & (PK - 1)) >> BSH
        for q in range(16):
            b = bv[q]
            hist[b] = hist[b] + 1
        return cy

    lax.fori_loop(0, S // 16, count_loop, 0)

    def emit(g, cy):
        vv = jnp.zeros((16,), i32)
        for q in range(16):
            vv = jnp.where(iota == q, jnp.full((16,), hist[g * 16 + q], i32), vv)
        stage[pl.ds(0, 16)] = vv
        oidx = (g * 16 + iota) * NW + w
        pltpu.sync_copy(stage.at[pl.ds(0, 16)], cnts.at[oidx])
        return cy

    lax.fori_loop(0, BINS // 16, emit, 0)


def _make_count_kernel():
    mesh = plsc.VectorSubcoreMesh(core_axis_name="c", subcore_axis_name="s")
    return pl.kernel(
        _count_body,
        out_type=jax.ShapeDtypeStruct((BINS * NW,), i32),
        mesh=mesh,
        scratch_types=[
            pltpu.VMEM((S,), i32),
            pltpu.VMEM((16,), i32),
            pltpu.SMEM((BINS,), i32),
            pltpu.SemaphoreType.DMA,
        ],
    )


def _scatter_body(epk, cnts, bpack, bstarts, bends, spk, cbuf, stage, zi,
                  hist, sem):
    c = lax.axis_index("c")
    s = lax.axis_index("s")
    w = c * 16 + s
    iota = lax.iota(i32, 16)
    pltpu.sync_copy(epk.at[pl.ds(pl.multiple_of(w * S, 32), S)], spk)
    pltpu.sync_copy(cnts, cbuf)

    def scan(b, gstart):
        v0 = cbuf[pl.ds(pl.multiple_of(b * NW, 32), 16)]
        v1 = cbuf[pl.ds(pl.multiple_of(b * NW, 32) + 16, 16)]
        tot = jnp.int32(0)
        woff = jnp.int32(0)
        for q in range(16):
            cq = v0[q]
            woff = woff + jnp.where(jnp.int32(q) < w, cq, 0)
            tot = tot + cq
        for q in range(16):
            cq = v1[q]
            woff = woff + jnp.where(jnp.int32(16 + q) < w, cq, 0)
            tot = tot + cq
        hist[b] = gstart + woff
        hist[BINS + b] = gstart + tot
        return gstart + ((tot + 15) >> 4 << 4)

    lax.fori_loop(0, BINS, scan, jnp.int32(0))

    @pl.when(w == 0)
    def _emit_bounds():
        def emit(g, cy):
            sv = jnp.zeros((16,), i32)
            evv = jnp.zeros((16,), i32)
            for q in range(16):
                st_q = hist[g * 16 + q]
                en_q = hist[BINS + g * 16 + q]
                sv = jnp.where(iota == q, jnp.full((16,), st_q, i32), sv)
                evv = jnp.where(iota == q, jnp.full((16,), en_q, i32), evv)
            stage[pl.ds(0, 16)] = sv
            stage[pl.ds(16, 16)] = evv
            pltpu.sync_copy(stage.at[pl.ds(0, 16)],
                            bstarts.at[pl.ds(pl.multiple_of(g * 16, 16), 16)])
            pltpu.sync_copy(stage.at[pl.ds(16, 16)],
                            bends.at[pl.ds(pl.multiple_of(g * 16, 16), 16)])
            return cy
        lax.fori_loop(0, BINS // 16, emit, 0)

    def scat_body(k, cy):
        bv = (spk[pl.ds(k * 16, 16)] & (PK - 1)) >> BSH
        posv = jnp.zeros((16,), i32)
        for q in range(16):
            b = bv[q]
            p = hist[b]
            hist[b] = p + 1
            posv = jnp.where(iota == q, jnp.full((16,), p, i32), posv)
        pltpu.sync_copy(spk.at[pl.ds(k * 16, 16)], bpack.at[posv])
        return cy

    lax.fori_loop(0, S // 16, scat_body, 0)


def _make_scatter_kernel():
    mesh = plsc.VectorSubcoreMesh(core_axis_name="c", subcore_axis_name="s")
    return pl.kernel(
        _scatter_body,
        out_type=(
            jax.ShapeDtypeStruct((BPK,), i32),
            jax.ShapeDtypeStruct((BINS,), i32),
            jax.ShapeDtypeStruct((BINS,), i32),
        ),
        mesh=mesh,
        scratch_types=[
            pltpu.VMEM((S,), i32),
            pltpu.VMEM((BINS * NW,), i32),
            pltpu.VMEM((32,), i32),
            pltpu.VMEM((16,), i32),
            pltpu.SMEM((2 * BINS,), i32),
            pltpu.SemaphoreType.DMA,
        ],
    )


def _layer_body(xl, xr, bpack, bstarts, bends, atth, pah, pbh, hout,
                attv, pav, pbv, xlr0, xrr0, xlr1, xrr1, acc, ebuf, rvb,
                stv, smb, sem0, sem1):
    c = lax.axis_index("c")
    s = lax.axis_index("s")
    tid = c * 16 + s
    iota = lax.iota(i32, 16)
    zf = jnp.zeros((16,), f32)

    pltpu.sync_copy(atth, attv)
    pltpu.sync_copy(pah, pav)
    pltpu.sync_copy(pbh, pbv)

    b0 = tid * BPT
    off8 = pl.multiple_of((b0 >> 3) << 3, 8)
    shift = b0 - off8
    pltpu.sync_copy(bstarts.at[pl.ds(off8, 16)], stv.at[pl.ds(0, 16)])
    pltpu.sync_copy(bends.at[pl.ds(off8, 16)], stv.at[pl.ds(16, 16)])
    sv = stv[pl.ds(0, 16)]
    evv = stv[pl.ds(16, 16)]
    for q in range(16):
        smb[q] = sv[q]
        smb[16 + q] = evv[q]

    def issue(pos16, bxl, bxr, semx):
        pkv = ebuf[pl.ds(pos16, 16)]
        srcv = jnp.clip(pkv >> 14, 0, NPAD - 1)
        dstv = jnp.minimum(pkv & (PK - 1), NPAD - 1)
        pltpu.async_copy(xl.at[srcv], bxl, semx)
        pltpu.async_copy(xr.at[dstv], bxr, semx)

    def wait(bxl, bxr, semx):
        pltpu.make_async_copy(xl.at[iota], bxl, semx).wait()
        pltpu.make_async_copy(xr.at[iota], bxr, semx).wait()

    def bin_body(bi, carry):
        bn = tid * BPT + bi
        base = bn * BNB
        st = smb[shift + bi]
        en = smb[16 + shift + bi]
        cnt = en - st

        def zb(row, cy):
            for g in range(AW // 16):
                acc[row, pl.ds(g * 16, 16)] = zf
            return cy
        lax.fori_loop(0, BNB, zb, 0)

        def compute(cidx, local16, bxl, bxr):
            pkv = ebuf[pl.ds(local16, 16)]
            dstv = jnp.minimum(pkv & (PK - 1), NPAD - 1)
            rvb[pl.ds(0, 16)] = dstv - base
            nv = jnp.clip(cnt - cidx * 16, 0, 16)

            def edge_body(e, cy3):
                row = rvb[pl.ds(e, 16)][0]
                pvs = []
                for h in range(H):
                    part = zf
                    for j in range(C // 16):
                        col = h * C + j * 16
                        t = bxl[e, pl.ds(col, 16)] + bxr[e, pl.ds(col, 16)]
                        t = jnp.maximum(t, 0.2 * t)
                        part = part + t * attv[pl.ds(col, 16)]
                    tree = [part[q] for q in range(16)]
                    while len(tree) > 1:
                        tree = [tree[2 * q] + tree[2 * q + 1]
                                for q in range(len(tree) // 2)]
                    pv = jnp.exp(jnp.full((16,), tree[0], f32))
                    pvs.append(pv)
                    for j in range(C // 16):
                        col = h * C + j * 16
                        plsc.addupdate(acc.at[row, pl.ds(col, 16)],
                                       pv * bxl[e, pl.ds(col, 16)])
                ptail = jnp.where(iota == 0, pvs[0],
                         jnp.where(iota == 1, pvs[1],
                          jnp.where(iota == 2, pvs[2],
                           jnp.where(iota == 3, pvs[3], zf))))
                plsc.addupdate(acc.at[row, pl.ds(HD, 16)], ptail)
                return cy3

            lax.fori_loop(0, nv, edge_body, 0)

        nsb = (cnt + SBE - 1) // SBE

        def sb_body(sb, cy):
            sbase = pl.multiple_of(st + sb * SBE, 16)
            rem = cnt - sb * SBE
            nblk = jnp.minimum((rem + 255) // 256, SBE // 256)

            def ld(t2, cy2):
                o = pl.multiple_of(t2 * 256, 256)
                pltpu.sync_copy(bpack.at[pl.ds(sbase + o, 256)],
                                ebuf.at[pl.ds(o, 256)])
                return cy2
            lax.fori_loop(0, nblk, ld, 0)

            nchk = jnp.clip((rem + 15) // 16, 0, SBE // 16)

            @pl.when(nchk > 0)
            def _pro():
                issue(0, xlr0, xrr0, sem0)

            def pair_body(m, cy2):
                c0 = 2 * m
                c1 = 2 * m + 1
                cond1 = c1 < nchk
                cond2 = c1 + 1 < nchk
                wait(xlr0, xrr0, sem0)

                @pl.when(cond1)
                def _i1():
                    issue(c1 * 16, xlr1, xrr1, sem1)

                compute(sb * (SBE // 16) + c0, c0 * 16, xlr0, xrr0)

                @pl.when(cond2)
                def _i2():
                    issue((c1 + 1) * 16, xlr0, xrr0, sem0)

                @pl.when(cond1)
                def _c1():
                    wait(xlr1, xrr1, sem1)
                    compute(sb * (SBE // 16) + c1, c1 * 16, xlr1, xrr1)

                return cy2

            lax.fori_loop(0, (nchk + 1) // 2, pair_body, 0)
            return cy

        lax.fori_loop(0, nsb, sb_body, 0)

        def nr(row, cy):
            dinvv = 1.0 / (acc[row, pl.ds(HD, 16)] + 1e-16)
            for h in range(H):
                dv = jnp.full((16,), dinvv[h], f32)
                for j in range(C // 16):
                    col = h * C + j * 16
                    o = acc[row, pl.ds(col, 16)] * dv * pav[pl.ds(col, 16)] \
                        + pbv[pl.ds(col, 16)]
                    acc[row, pl.ds(col, 16)] = jnp.maximum(o, 0.0)
            return cy
        lax.fori_loop(0, BNB, nr, 0)

        nrows = jnp.clip(N - base, 0, BNB)
        nchw = nrows // 16

        def wr(q, cy):
            pltpu.sync_copy(acc.at[pl.ds(q * 16, 16)],
                            hout.at[pl.ds(base + q * 16, 16)])
            return cy
        lax.fori_loop(0, nchw, wr, 0)
        return carry

    lax.fori_loop(0, BPT, bin_body, 0)


def _make_layer_kernel():
    mesh = plsc.VectorSubcoreMesh(core_axis_name="c", subcore_axis_name="s")
    return pl.kernel(
        _layer_body,
        out_type=jax.ShapeDtypeStruct((NPAD, AW), f32),
        mesh=mesh,
        scratch_types=[
            pltpu.VMEM((HD,), f32),
            pltpu.VMEM((HD,), f32),
            pltpu.VMEM((HD,), f32),
            pltpu.VMEM((16, HD), f32),
            pltpu.VMEM((16, HD), f32),
            pltpu.VMEM((16, HD), f32),
            pltpu.VMEM((16, HD), f32),
            pltpu.VMEM((BNB, AW), f32),
            pltpu.VMEM((SBE,), i32),
            pltpu.VMEM((32,), i32),
            pltpu.VMEM((32,), i32),
            pltpu.SMEM((32,), i32),
            pltpu.SemaphoreType.DMA,
            pltpu.SemaphoreType.DMA,
        ],
    )


def kernel(x, edge_index, Wl0, bl0, Wr0, br0, att0, bias0, g0, be0, rm0, rv0,
           Wl1, bl1, Wr1, br1, att1, bias1, g1, be1, rm1, rv1, W1, b1, W2, b2):
    loop = jnp.arange(N, dtype=edge_index.dtype)
    srcp = jnp.concatenate([edge_index[0], loop,
                            jnp.zeros((EPT - E - N,), i32)])
    dstp = jnp.concatenate([edge_index[1], loop,
                            jnp.full((EPT - E - N,), NPAD - 1, i32)])
    epk = srcp * PK + dstp

    cnts = _make_count_kernel()(epk)
    bpack, bstarts, bends = _make_scatter_kernel()(epk, cnts)

    s0 = g0 * lax.rsqrt(rv0 + 1e-5)
    pb0 = bias0 * s0 + (be0 - rm0 * s0)
    s1 = g1 * lax.rsqrt(rv1 + 1e-5)
    pb1 = bias1 * s1 + (be1 - rm1 * s1)

    layer = _make_layer_kernel()

    xp = jnp.pad(x, ((0, NPAD - N), (0, 0)))
    xl0, xr0 = _mm2(xp, Wl0, bl0, Wr0, br0)
    h = layer(xl0, xr0, bpack, bstarts, bends, att0.reshape(HD), s0, pb0)

    hp = jnp.pad(h[:N, :HD], ((0, NPAD - N), (0, 0)))
    xl1, xr1 = _mm2(hp, Wl1, bl1, Wr1, br1)
    h = layer(xl1, xr1, bpack, bstarts, bends, att1.reshape(HD), s1, pb1)

    hp = jnp.pad(h[:N, :HD], ((0, NPAD - N), (0, 0)))
    h = _mm(hp, W1, b1, act="relu")
    h = _mm(h, W2, b2)[:N]
    return h


# speedup vs baseline: 7.7975x; 1.1250x over previous
import functools

import jax
import jax.numpy as jnp
from jax import lax
from jax.experimental import pallas as pl
from jax.experimental.pallas import tpu as pltpu
from jax.experimental.pallas import tpu_sc as plsc

i32 = jnp.int32
f32 = jnp.float32

N = 10000
E = 160000
D = 256
H = 4
C = 256
HD = H * C
OUT = 128

NW = 32
S = 5344
EPT = NW * S
BINS = 320
BSH = 5
BNB = 32
PK = 16384
NPAD = BINS * BNB
AW = HD + 16
BPT = BINS // NW
BPK = EPT + BINS * 16 + 512
SBE = 4096
BM = 512


def _mm_kernel(x_ref, w_ref, b_ref, o_ref, *, act):
    acc = jnp.dot(x_ref[...], w_ref[...], preferred_element_type=f32)
    acc = acc + b_ref[...][None, :]
    if act == "relu":
        acc = jnp.maximum(acc, 0.0)
    o_ref[...] = acc


def _mm(x, w, b, act=None):
    M, K = x.shape
    Nout = w.shape[1]
    return pl.pallas_call(
        functools.partial(_mm_kernel, act=act),
        grid=(M // BM,),
        in_specs=[
            pl.BlockSpec((BM, K), lambda i: (i, 0)),
            pl.BlockSpec((K, Nout), lambda i: (0, 0)),
            pl.BlockSpec((Nout,), lambda i: (0,)),
        ],
        out_specs=pl.BlockSpec((BM, Nout), lambda i: (i, 0)),
        out_shape=jax.ShapeDtypeStruct((M, Nout), f32),
    )(x, w, b)


def _mm2_kernel(x_ref, wl_ref, bl_ref, wr_ref, br_ref, ol_ref, or_ref):
    xv = x_ref[...]
    ol_ref[...] = jnp.dot(xv, wl_ref[...], preferred_element_type=f32) + bl_ref[...][None, :]
    or_ref[...] = jnp.dot(xv, wr_ref[...], preferred_element_type=f32) + br_ref[...][None, :]


def _mm2(x, wl, bl, wr, br):
    M, K = x.shape
    return pl.pallas_call(
        _mm2_kernel,
        grid=(M // BM,),
        in_specs=[
            pl.BlockSpec((BM, K), lambda i: (i, 0)),
            pl.BlockSpec((K, HD), lambda i: (0, 0)),
            pl.BlockSpec((HD,), lambda i: (0,)),
            pl.BlockSpec((K, HD), lambda i: (0, 0)),
            pl.BlockSpec((HD,), lambda i: (0,)),
        ],
        out_specs=[
            pl.BlockSpec((BM, HD), lambda i: (i, 0)),
            pl.BlockSpec((BM, HD), lambda i: (i, 0)),
        ],
        out_shape=[
            jax.ShapeDtypeStruct((M, HD), f32),
            jax.ShapeDtypeStruct((M, HD), f32),
        ],
    )(x, wl, bl, wr, br)


def _count_body(epk, cnts, spk, stage, hist, sem):
    c = lax.axis_index("c")
    s = lax.axis_index("s")
    w = c * 16 + s
    iota = lax.iota(i32, 16)
    pltpu.sync_copy(epk.at[pl.ds(pl.multiple_of(w * S, 32), S)], spk)

    for r in range(BINS):
        hist[r] = jnp.int32(0)

    def count_loop(k, cy):
        bv = (spk[pl.ds(k * 16, 16)] & (PK - 1)) >> BSH
        for q in range(16):
            b = bv[q]
            hist[b] = hist[b] + 1
        return cy

    lax.fori_loop(0, S // 16, count_loop, 0)

    def emit(g, cy):
        vv = jnp.zeros((16,), i32)
        for q in range(16):
            vv = jnp.where(iota == q, jnp.full((16,), hist[g * 16 + q], i32), vv)
        stage[pl.ds(0, 16)] = vv
        oidx = (g * 16 + iota) * NW + w
        pltpu.sync_copy(stage.at[pl.ds(0, 16)], cnts.at[oidx])
        return cy

    lax.fori_loop(0, BINS // 16, emit, 0)


def _make_count_kernel():
    mesh = plsc.VectorSubcoreMesh(core_axis_name="c", subcore_axis_name="s")
    return pl.kernel(
        _count_body,
        out_type=jax.ShapeDtypeStruct((BINS * NW,), i32),
        mesh=mesh,
        scratch_types=[
            pltpu.VMEM((S,), i32),
            pltpu.VMEM((16,), i32),
            pltpu.SMEM((BINS,), i32),
            pltpu.SemaphoreType.DMA,
        ],
    )


def _scatter_body(epk, cnts, bpack, bstarts, bends, spk, cbuf, stage, zi,
                  hist, sem):
    c = lax.axis_index("c")
    s = lax.axis_index("s")
    w = c * 16 + s
    iota = lax.iota(i32, 16)
    pltpu.sync_copy(epk.at[pl.ds(pl.multiple_of(w * S, 32), S)], spk)
    pltpu.sync_copy(cnts, cbuf)

    def scan(b, gstart):
        v0 = cbuf[pl.ds(pl.multiple_of(b * NW, 32), 16)]
        v1 = cbuf[pl.ds(pl.multiple_of(b * NW, 32) + 16, 16)]
        tot = jnp.int32(0)
        woff = jnp.int32(0)
        for q in range(16):
            cq = v0[q]
            woff = woff + jnp.where(jnp.int32(q) < w, cq, 0)
            tot = tot + cq
        for q in range(16):
            cq = v1[q]
            woff = woff + jnp.where(jnp.int32(16 + q) < w, cq, 0)
            tot = tot + cq
        hist[b] = gstart + woff
        hist[BINS + b] = gstart + tot
        return gstart + ((tot + 15) >> 4 << 4)

    lax.fori_loop(0, BINS, scan, jnp.int32(0))

    @pl.when(w == 0)
    def _emit_bounds():
        def emit(g, cy):
            sv = jnp.zeros((16,), i32)
            evv = jnp.zeros((16,), i32)
            for q in range(16):
                st_q = hist[g * 16 + q]
                en_q = hist[BINS + g * 16 + q]
                sv = jnp.where(iota == q, jnp.full((16,), st_q, i32), sv)
                evv = jnp.where(iota == q, jnp.full((16,), en_q, i32), evv)
            stage[pl.ds(0, 16)] = sv
            stage[pl.ds(16, 16)] = evv
            pltpu.sync_copy(stage.at[pl.ds(0, 16)],
                            bstarts.at[pl.ds(pl.multiple_of(g * 16, 16), 16)])
            pltpu.sync_copy(stage.at[pl.ds(16, 16)],
                            bends.at[pl.ds(pl.multiple_of(g * 16, 16), 16)])
            return cy
        lax.fori_loop(0, BINS // 16, emit, 0)

    def scat_body(k, cy):
        bv = (spk[pl.ds(k * 16, 16)] & (PK - 1)) >> BSH
        posv = jnp.zeros((16,), i32)
        for q in range(16):
            b = bv[q]
            p = hist[b]
            hist[b] = p + 1
            posv = jnp.where(iota == q, jnp.full((16,), p, i32), posv)
        pltpu.sync_copy(spk.at[pl.ds(k * 16, 16)], bpack.at[posv])
        return cy

    lax.fori_loop(0, S // 16, scat_body, 0)


def _make_scatter_kernel():
    mesh = plsc.VectorSubcoreMesh(core_axis_name="c", subcore_axis_name="s")
    return pl.kernel(
        _scatter_body,
        out_type=(
            jax.ShapeDtypeStruct((BPK,), i32),
            jax.ShapeDtypeStruct((BINS,), i32),
            jax.ShapeDtypeStruct((BINS,), i32),
        ),
        mesh=mesh,
        scratch_types=[
            pltpu.VMEM((S,), i32),
            pltpu.VMEM((BINS * NW,), i32),
            pltpu.VMEM((32,), i32),
            pltpu.VMEM((16,), i32),
            pltpu.SMEM((2 * BINS,), i32),
            pltpu.SemaphoreType.DMA,
        ],
    )


def _layer_body(xl, xr, bpack, bstarts, bends, atth, pah, pbh, hout,
                attv, pav, pbv, xlr0, xrr0, xlr1, xrr1, acc, ebuf, rvb,
                stv, smb, sem0, sem1):
    c = lax.axis_index("c")
    s = lax.axis_index("s")
    tid = c * 16 + s
    iota = lax.iota(i32, 16)
    zf = jnp.zeros((16,), f32)

    pltpu.sync_copy(atth, attv)
    pltpu.sync_copy(pah, pav)
    pltpu.sync_copy(pbh, pbv)

    b0 = tid * BPT
    off8 = pl.multiple_of((b0 >> 3) << 3, 8)
    shift = b0 - off8
    pltpu.sync_copy(bstarts.at[pl.ds(off8, 16)], stv.at[pl.ds(0, 16)])
    pltpu.sync_copy(bends.at[pl.ds(off8, 16)], stv.at[pl.ds(16, 16)])
    sv = stv[pl.ds(0, 16)]
    evv = stv[pl.ds(16, 16)]
    for q in range(16):
        smb[q] = sv[q]
        smb[16 + q] = evv[q]

    def issue(pos16, bxl, bxr, semx):
        pkv = ebuf[pl.ds(pos16, 16)]
        srcv = jnp.clip(pkv >> 14, 0, NPAD - 1)
        dstv = jnp.minimum(pkv & (PK - 1), NPAD - 1)
        pltpu.async_copy(xl.at[srcv], bxl, semx)
        pltpu.async_copy(xr.at[dstv], bxr, semx)

    def wait(bxl, bxr, semx):
        pltpu.make_async_copy(xl.at[iota], bxl, semx).wait()
        pltpu.make_async_copy(xr.at[iota], bxr, semx).wait()

    def bin_body(bi, carry):
        bn = tid * BPT + bi
        base = bn * BNB
        st = smb[shift + bi]
        en = smb[16 + shift + bi]
        cnt = en - st

        def zb(row, cy):
            for g in range(AW // 16):
                acc[row, pl.ds(g * 16, 16)] = zf
            return cy
        lax.fori_loop(0, BNB, zb, 0)

        def compute(cidx, local16, bxl, bxr):
            pkv = ebuf[pl.ds(local16, 16)]
            dstv = jnp.minimum(pkv & (PK - 1), NPAD - 1)
            rvb[pl.ds(0, 16)] = dstv - base
            nv = jnp.clip(cnt - cidx * 16, 0, 16)

            def tree_sum(part):
                tree = [part[q] for q in range(16)]
                while len(tree) > 1:
                    tree = [tree[2 * q] + tree[2 * q + 1]
                            for q in range(len(tree) // 2)]
                return tree[0]

            def ptail_of(pvs):
                return jnp.where(iota == 0, pvs[0],
                        jnp.where(iota == 1, pvs[1],
                         jnp.where(iota == 2, pvs[2],
                          jnp.where(iota == 3, pvs[3], zf))))

            def pair_edges(m, cy3):
                e0 = 2 * m
                e1 = 2 * m + 1
                row0 = rvb[pl.ds(e0, 16)][0]
                row1 = rvb[pl.ds(e1, 16)][0]
                pvs0 = []
                pvs1 = []
                for h in range(H):
                    p0 = zf
                    p1 = zf
                    for j in range(C // 16):
                        col = h * C + j * 16
                        av = attv[pl.ds(col, 16)]
                        t0 = bxl[e0, pl.ds(col, 16)] + bxr[e0, pl.ds(col, 16)]
                        p0 = p0 + jnp.maximum(t0, 0.2 * t0) * av
                        t1 = bxl[e1, pl.ds(col, 16)] + bxr[e1, pl.ds(col, 16)]
                        p1 = p1 + jnp.maximum(t1, 0.2 * t1) * av
                    pv0 = jnp.exp(jnp.full((16,), tree_sum(p0), f32))
                    pv1 = jnp.exp(jnp.full((16,), tree_sum(p1), f32))
                    pvs0.append(pv0)
                    pvs1.append(pv1)
                    for j in range(C // 16):
                        col = h * C + j * 16
                        plsc.addupdate(acc.at[row0, pl.ds(col, 16)],
                                       pv0 * bxl[e0, pl.ds(col, 16)])
                        plsc.addupdate(acc.at[row1, pl.ds(col, 16)],
                                       pv1 * bxl[e1, pl.ds(col, 16)])
                plsc.addupdate(acc.at[row0, pl.ds(HD, 16)], ptail_of(pvs0))
                plsc.addupdate(acc.at[row1, pl.ds(HD, 16)], ptail_of(pvs1))
                return cy3

            def edge_body(e, cy3):
                row = rvb[pl.ds(e, 16)][0]
                pvs = []
                for h in range(H):
                    part = zf
                    for j in range(C // 16):
                        col = h * C + j * 16
                        t = bxl[e, pl.ds(col, 16)] + bxr[e, pl.ds(col, 16)]
                        part = part + jnp.maximum(t, 0.2 * t) * attv[pl.ds(col, 16)]
                    pv = jnp.exp(jnp.full((16,), tree_sum(part), f32))
                    pvs.append(pv)
                    for j in range(C // 16):
                        col = h * C + j * 16
                        plsc.addupdate(acc.at[row, pl.ds(col, 16)],
                                       pv * bxl[e, pl.ds(col, 16)])
                plsc.addupdate(acc.at[row, pl.ds(HD, 16)], ptail_of(pvs))
                return cy3

            npair = nv // 2
            lax.fori_loop(0, npair, pair_edges, 0)
            lax.fori_loop(npair * 2, nv, edge_body, 0)

        nsb = (cnt + SBE - 1) // SBE

        def sb_body(sb, cy):
            sbase = pl.multiple_of(st + sb * SBE, 16)
            rem = cnt - sb * SBE
            nblk = jnp.minimum((rem + 255) // 256, SBE // 256)

            def ld(t2, cy2):
                o = pl.multiple_of(t2 * 256, 256)
                pltpu.sync_copy(bpack.at[pl.ds(sbase + o, 256)],
                                ebuf.at[pl.ds(o, 256)])
                return cy2
            lax.fori_loop(0, nblk, ld, 0)

            nchk = jnp.clip((rem + 15) // 16, 0, SBE // 16)

            @pl.when(nchk > 0)
            def _pro():
                issue(0, xlr0, xrr0, sem0)

            def pair_body(m, cy2):
                c0 = 2 * m
                c1 = 2 * m + 1
                cond1 = c1 < nchk
                cond2 = c1 + 1 < nchk
                wait(xlr0, xrr0, sem0)

                @pl.when(cond1)
                def _i1():
                    issue(c1 * 16, xlr1, xrr1, sem1)

                compute(sb * (SBE // 16) + c0, c0 * 16, xlr0, xrr0)

                @pl.when(cond2)
                def _i2():
                    issue((c1 + 1) * 16, xlr0, xrr0, sem0)

                @pl.when(cond1)
                def _c1():
                    wait(xlr1, xrr1, sem1)
                    compute(sb * (SBE // 16) + c1, c1 * 16, xlr1, xrr1)

                return cy2

            lax.fori_loop(0, (nchk + 1) // 2, pair_body, 0)
            return cy

        lax.fori_loop(0, nsb, sb_body, 0)

        def nr(row, cy):
            dinvv = 1.0 / (acc[row, pl.ds(HD, 16)] + 1e-16)
            for h in range(H):
                dv = jnp.full((16,), dinvv[h], f32)
                for j in range(C // 16):
                    col = h * C + j * 16
                    o = acc[row, pl.ds(col, 16)] * dv * pav[pl.ds(col, 16)] \
                        + pbv[pl.ds(col, 16)]
                    acc[row, pl.ds(col, 16)] = jnp.maximum(o, 0.0)
            return cy
        lax.fori_loop(0, BNB, nr, 0)

        nrows = jnp.clip(N - base, 0, BNB)
        nchw = nrows // 16

        def wr(q, cy):
            pltpu.sync_copy(acc.at[pl.ds(q * 16, 16)],
                            hout.at[pl.ds(base + q * 16, 16)])
            return cy
        lax.fori_loop(0, nchw, wr, 0)
        return carry

    lax.fori_loop(0, BPT, bin_body, 0)


def _make_layer_kernel():
    mesh = plsc.VectorSubcoreMesh(core_axis_name="c", subcore_axis_name="s")
    return pl.kernel(
        _layer_body,
        out_type=jax.ShapeDtypeStruct((NPAD, AW), f32),
        mesh=mesh,
        scratch_types=[
            pltpu.VMEM((HD,), f32),
            pltpu.VMEM((HD,), f32),
            pltpu.VMEM((HD,), f32),
            pltpu.VMEM((16, HD), f32),
            pltpu.VMEM((16, HD), f32),
            pltpu.VMEM((16, HD), f32),
            pltpu.VMEM((16, HD), f32),
            pltpu.VMEM((BNB, AW), f32),
            pltpu.VMEM((SBE,), i32),
            pltpu.VMEM((32,), i32),
            pltpu.VMEM((32,), i32),
            pltpu.SMEM((32,), i32),
            pltpu.SemaphoreType.DMA,
            pltpu.SemaphoreType.DMA,
        ],
    )


def kernel(x, edge_index, Wl0, bl0, Wr0, br0, att0, bias0, g0, be0, rm0, rv0,
           Wl1, bl1, Wr1, br1, att1, bias1, g1, be1, rm1, rv1, W1, b1, W2, b2):
    loop = jnp.arange(N, dtype=edge_index.dtype)
    srcp = jnp.concatenate([edge_index[0], loop,
                            jnp.zeros((EPT - E - N,), i32)])
    dstp = jnp.concatenate([edge_index[1], loop,
                            jnp.full((EPT - E - N,), NPAD - 1, i32)])
    epk = srcp * PK + dstp

    cnts = _make_count_kernel()(epk)
    bpack, bstarts, bends = _make_scatter_kernel()(epk, cnts)

    s0 = g0 * lax.rsqrt(rv0 + 1e-5)
    pb0 = bias0 * s0 + (be0 - rm0 * s0)
    s1 = g1 * lax.rsqrt(rv1 + 1e-5)
    pb1 = bias1 * s1 + (be1 - rm1 * s1)

    layer = _make_layer_kernel()

    xp = jnp.pad(x, ((0, NPAD - N), (0, 0)))
    xl0, xr0 = _mm2(xp, Wl0, bl0, Wr0, br0)
    h = layer(xl0, xr0, bpack, bstarts, bends, att0.reshape(HD), s0, pb0)

    hp = jnp.pad(h[:N, :HD], ((0, NPAD - N), (0, 0)))
    xl1, xr1 = _mm2(hp, Wl1, bl1, Wr1, br1)
    h = layer(xl1, xr1, bpack, bstarts, bends, att1.reshape(HD), s1, pb1)

    hp = jnp.pad(h[:N, :HD], ((0, NPAD - N), (0, 0)))
    h = _mm(hp, W1, b1, act="relu")
    h = _mm(h, W2, b2)[:N]
    return h


# speedup vs baseline: 13.8022x; 1.7701x over previous
import functools

import jax
import jax.numpy as jnp
from jax import lax
from jax.experimental import pallas as pl
from jax.experimental.pallas import tpu as pltpu
from jax.experimental.pallas import tpu_sc as plsc

i32 = jnp.int32
f32 = jnp.float32

N = 10000
E = 160000
D = 256
H = 4
C = 256
HD = H * C
OUT = 128

NW = 32
S = 5344
EPT = NW * S
BINS = 320
BSH = 5
BNB = 32
PK = 16384
NPAD = BINS * BNB
AW = HD + 16
BPT = BINS // NW
BPK = EPT + BINS * 16 + 512
SBE = 4096
BM = 512


def _mm_kernel(x_ref, w_ref, b_ref, o_ref, *, act):
    acc = jnp.dot(x_ref[...], w_ref[...], preferred_element_type=f32)
    acc = acc + b_ref[...][None, :]
    if act == "relu":
        acc = jnp.maximum(acc, 0.0)
    o_ref[...] = acc


def _mm(x, w, b, act=None):
    M, K = x.shape
    Nout = w.shape[1]
    return pl.pallas_call(
        functools.partial(_mm_kernel, act=act),
        grid=(M // BM,),
        in_specs=[
            pl.BlockSpec((BM, K), lambda i: (i, 0)),
            pl.BlockSpec((K, Nout), lambda i: (0, 0)),
            pl.BlockSpec((Nout,), lambda i: (0,)),
        ],
        out_specs=pl.BlockSpec((BM, Nout), lambda i: (i, 0)),
        out_shape=jax.ShapeDtypeStruct((M, Nout), f32),
    )(x, w, b)


def _mm2_kernel(x_ref, wl_ref, bl_ref, wr_ref, br_ref, ol_ref, or_ref):
    xv = x_ref[...]
    ol_ref[...] = jnp.dot(xv, wl_ref[...], preferred_element_type=f32) + bl_ref[...][None, :]
    or_ref[...] = jnp.dot(xv, wr_ref[...], preferred_element_type=f32) + br_ref[...][None, :]


def _mm2(x, wl, bl, wr, br):
    M, K = x.shape
    return pl.pallas_call(
        _mm2_kernel,
        grid=(M // BM,),
        in_specs=[
            pl.BlockSpec((BM, K), lambda i: (i, 0)),
            pl.BlockSpec((K, HD), lambda i: (0, 0)),
            pl.BlockSpec((HD,), lambda i: (0,)),
            pl.BlockSpec((K, HD), lambda i: (0, 0)),
            pl.BlockSpec((HD,), lambda i: (0,)),
        ],
        out_specs=[
            pl.BlockSpec((BM, HD), lambda i: (i, 0)),
            pl.BlockSpec((BM, HD), lambda i: (i, 0)),
        ],
        out_shape=[
            jax.ShapeDtypeStruct((M, HD), f32),
            jax.ShapeDtypeStruct((M, HD), f32),
        ],
    )(x, wl, bl, wr, br)


def _count_body(epk, cnts, spk, stage, hist, sem):
    c = lax.axis_index("c")
    s = lax.axis_index("s")
    w = c * 16 + s
    iota = lax.iota(i32, 16)
    pltpu.sync_copy(epk.at[pl.ds(pl.multiple_of(w * S, 32), S)], spk)

    for r in range(BINS):
        hist[r] = jnp.int32(0)

    def count_loop(k, cy):
        bv = (spk[pl.ds(k * 16, 16)] & (PK - 1)) >> BSH
        for q in range(16):
            b = bv[q]
            hist[b] = hist[b] + 1
        return cy

    lax.fori_loop(0, S // 16, count_loop, 0)

    def emit(g, cy):
        vv = jnp.zeros((16,), i32)
        for q in range(16):
            vv = jnp.where(iota == q, jnp.full((16,), hist[g * 16 + q], i32), vv)
        stage[pl.ds(0, 16)] = vv
        oidx = (g * 16 + iota) * NW + w
        pltpu.sync_copy(stage.at[pl.ds(0, 16)], cnts.at[oidx])
        return cy

    lax.fori_loop(0, BINS // 16, emit, 0)


def _make_count_kernel():
    mesh = plsc.VectorSubcoreMesh(core_axis_name="c", subcore_axis_name="s")
    return pl.kernel(
        _count_body,
        out_type=jax.ShapeDtypeStruct((BINS * NW,), i32),
        mesh=mesh,
        scratch_types=[
            pltpu.VMEM((S,), i32),
            pltpu.VMEM((16,), i32),
            pltpu.SMEM((BINS,), i32),
            pltpu.SemaphoreType.DMA,
        ],
    )


def _scatter_body(epk, cnts, bpack, bstarts, bends, spk, cbuf, stage, zi,
                  hist, sem):
    c = lax.axis_index("c")
    s = lax.axis_index("s")
    w = c * 16 + s
    iota = lax.iota(i32, 16)
    pltpu.sync_copy(epk.at[pl.ds(pl.multiple_of(w * S, 32), S)], spk)
    pltpu.sync_copy(cnts, cbuf)

    def scan(b, gstart):
        v0 = cbuf[pl.ds(pl.multiple_of(b * NW, 32), 16)]
        v1 = cbuf[pl.ds(pl.multiple_of(b * NW, 32) + 16, 16)]
        tot = jnp.int32(0)
        woff = jnp.int32(0)
        for q in range(16):
            cq = v0[q]
            woff = woff + jnp.where(jnp.int32(q) < w, cq, 0)
            tot = tot + cq
        for q in range(16):
            cq = v1[q]
            woff = woff + jnp.where(jnp.int32(16 + q) < w, cq, 0)
            tot = tot + cq
        hist[b] = gstart + woff
        hist[BINS + b] = gstart + tot
        return gstart + ((tot + 15) >> 4 << 4)

    lax.fori_loop(0, BINS, scan, jnp.int32(0))

    @pl.when(w == 0)
    def _emit_bounds():
        def emit(g, cy):
            sv = jnp.zeros((16,), i32)
            evv = jnp.zeros((16,), i32)
            for q in range(16):
                st_q = hist[g * 16 + q]
                en_q = hist[BINS + g * 16 + q]
                sv = jnp.where(iota == q, jnp.full((16,), st_q, i32), sv)
                evv = jnp.where(iota == q, jnp.full((16,), en_q, i32), evv)
            stage[pl.ds(0, 16)] = sv
            stage[pl.ds(16, 16)] = evv
            pltpu.sync_copy(stage.at[pl.ds(0, 16)],
                            bstarts.at[pl.ds(pl.multiple_of(g * 16, 16), 16)])
            pltpu.sync_copy(stage.at[pl.ds(16, 16)],
                            bends.at[pl.ds(pl.multiple_of(g * 16, 16), 16)])
            return cy
        lax.fori_loop(0, BINS // 16, emit, 0)

    def scat_body(k, cy):
        bv = (spk[pl.ds(k * 16, 16)] & (PK - 1)) >> BSH
        posv = jnp.zeros((16,), i32)
        for q in range(16):
            b = bv[q]
            p = hist[b]
            hist[b] = p + 1
            posv = jnp.where(iota == q, jnp.full((16,), p, i32), posv)
        pltpu.sync_copy(spk.at[pl.ds(k * 16, 16)], bpack.at[posv])
        return cy

    lax.fori_loop(0, S // 16, scat_body, 0)


def _make_scatter_kernel():
    mesh = plsc.VectorSubcoreMesh(core_axis_name="c", subcore_axis_name="s")
    return pl.kernel(
        _scatter_body,
        out_type=(
            jax.ShapeDtypeStruct((BPK,), i32),
            jax.ShapeDtypeStruct((BINS,), i32),
            jax.ShapeDtypeStruct((BINS,), i32),
        ),
        mesh=mesh,
        scratch_types=[
            pltpu.VMEM((S,), i32),
            pltpu.VMEM((BINS * NW,), i32),
            pltpu.VMEM((32,), i32),
            pltpu.VMEM((16,), i32),
            pltpu.SMEM((2 * BINS,), i32),
            pltpu.SemaphoreType.DMA,
        ],
    )


def _layer_body(xl, xr, bpack, bstarts, bends, atth, pah, pbh, hout,
                attv, pav, pbv, xlr0, xrr0, xlr1, xrr1, acc, ebuf, rvb,
                stv, smb, sem0, sem1):
    c = lax.axis_index("c")
    s = lax.axis_index("s")
    tid = c * 16 + s
    iota = lax.iota(i32, 16)
    zf = jnp.zeros((16,), f32)

    pltpu.sync_copy(atth, attv)
    pltpu.sync_copy(pah, pav)
    pltpu.sync_copy(pbh, pbv)

    b0 = tid * BPT
    off8 = pl.multiple_of((b0 >> 3) << 3, 8)
    shift = b0 - off8
    pltpu.sync_copy(bstarts.at[pl.ds(off8, 16)], stv.at[pl.ds(0, 16)])
    pltpu.sync_copy(bends.at[pl.ds(off8, 16)], stv.at[pl.ds(16, 16)])
    sv = stv[pl.ds(0, 16)]
    evv = stv[pl.ds(16, 16)]
    for q in range(16):
        smb[q] = sv[q]
        smb[16 + q] = evv[q]

    def issue(pos16, bxl, bxr, semx):
        pkv = ebuf[pl.ds(pos16, 16)]
        srcv = jnp.clip(pkv >> 14, 0, NPAD - 1)
        dstv = jnp.minimum(pkv & (PK - 1), NPAD - 1)
        pltpu.async_copy(xl.at[srcv], bxl, semx)
        pltpu.async_copy(xr.at[dstv], bxr, semx)

    def wait(bxl, bxr, semx):
        pltpu.make_async_copy(xl.at[iota], bxl, semx).wait()
        pltpu.make_async_copy(xr.at[iota], bxr, semx).wait()

    def bin_body(bi, carry):
        bn = tid * BPT + bi
        base = bn * BNB
        st = smb[shift + bi]
        en = smb[16 + shift + bi]
        cnt = en - st

        def zb(row, cy):
            for g in range(AW // 16):
                acc[row, pl.ds(g * 16, 16)] = zf
            return cy
        lax.fori_loop(0, BNB, zb, 0)

        def compute(cidx, local16, bxl, bxr):
            pkv = ebuf[pl.ds(local16, 16)]
            dstv = jnp.minimum(pkv & (PK - 1), NPAD - 1)
            rvb[pl.ds(0, 16)] = dstv - base
            nv = jnp.clip(cnt - cidx * 16, 0, 16)

            def tree_sum(part):
                tree = [part[q] for q in range(16)]
                while len(tree) > 1:
                    tree = [tree[2 * q] + tree[2 * q + 1]
                            for q in range(len(tree) // 2)]
                return tree[0]

            def ptail_of(pvs):
                return jnp.where(iota == 0, pvs[0],
                        jnp.where(iota == 1, pvs[1],
                         jnp.where(iota == 2, pvs[2],
                          jnp.where(iota == 3, pvs[3], zf))))

            def pair_edges(m, cy3):
                e0 = 2 * m
                e1 = 2 * m + 1
                row0 = rvb[pl.ds(e0, 16)][0]
                row1 = rvb[pl.ds(e1, 16)][0]
                pvs0 = []
                pvs1 = []
                for h in range(H):
                    p0 = zf
                    p1 = zf
                    xs0 = []
                    xs1 = []
                    for j in range(C // 16):
                        col = h * C + j * 16
                        av = attv[pl.ds(col, 16)]
                        x0 = bxl[e0, pl.ds(col, 16)]
                        x1 = bxl[e1, pl.ds(col, 16)]
                        xs0.append(x0)
                        xs1.append(x1)
                        t0 = x0 + bxr[e0, pl.ds(col, 16)]
                        p0 = p0 + jnp.maximum(t0, 0.2 * t0) * av
                        t1 = x1 + bxr[e1, pl.ds(col, 16)]
                        p1 = p1 + jnp.maximum(t1, 0.2 * t1) * av
                    pv0 = jnp.exp(jnp.full((16,), tree_sum(p0), f32))
                    pv1 = jnp.exp(jnp.full((16,), tree_sum(p1), f32))
                    pvs0.append(pv0)
                    pvs1.append(pv1)
                    for j in range(C // 16):
                        col = h * C + j * 16
                        plsc.addupdate(acc.at[row0, pl.ds(col, 16)],
                                       pv0 * xs0[j])
                        plsc.addupdate(acc.at[row1, pl.ds(col, 16)],
                                       pv1 * xs1[j])
                plsc.addupdate(acc.at[row0, pl.ds(HD, 16)], ptail_of(pvs0))
                plsc.addupdate(acc.at[row1, pl.ds(HD, 16)], ptail_of(pvs1))
                return cy3

            def edge_body(e, cy3):
                row = rvb[pl.ds(e, 16)][0]
                pvs = []
                for h in range(H):
                    part = zf
                    for j in range(C // 16):
                        col = h * C + j * 16
                        t = bxl[e, pl.ds(col, 16)] + bxr[e, pl.ds(col, 16)]
                        part = part + jnp.maximum(t, 0.2 * t) * attv[pl.ds(col, 16)]
                    pv = jnp.exp(jnp.full((16,), tree_sum(part), f32))
                    pvs.append(pv)
                    for j in range(C // 16):
                        col = h * C + j * 16
                        plsc.addupdate(acc.at[row, pl.ds(col, 16)],
                                       pv * bxl[e, pl.ds(col, 16)])
                plsc.addupdate(acc.at[row, pl.ds(HD, 16)], ptail_of(pvs))
                return cy3

            npair = nv // 2
            lax.fori_loop(0, npair, pair_edges, 0)
            lax.fori_loop(npair * 2, nv, edge_body, 0)

        nsb = (cnt + SBE - 1) // SBE

        def sb_body(sb, cy):
            sbase = pl.multiple_of(st + sb * SBE, 16)
            rem = cnt - sb * SBE
            nblk = jnp.minimum((rem + 255) // 256, SBE // 256)

            def ld(t2, cy2):
                o = pl.multiple_of(t2 * 256, 256)
                pltpu.sync_copy(bpack.at[pl.ds(sbase + o, 256)],
                                ebuf.at[pl.ds(o, 256)])
                return cy2
            lax.fori_loop(0, nblk, ld, 0)

            nchk = jnp.clip((rem + 15) // 16, 0, SBE // 16)

            @pl.when(nchk > 0)
            def _pro():
                issue(0, xlr0, xrr0, sem0)

            def pair_body(m, cy2):
                c0 = 2 * m
                c1 = 2 * m + 1
                cond1 = c1 < nchk
                cond2 = c1 + 1 < nchk
                wait(xlr0, xrr0, sem0)

                @pl.when(cond1)
                def _i1():
                    issue(c1 * 16, xlr1, xrr1, sem1)

                compute(sb * (SBE // 16) + c0, c0 * 16, xlr0, xrr0)

                @pl.when(cond2)
                def _i2():
                    issue((c1 + 1) * 16, xlr0, xrr0, sem0)

                @pl.when(cond1)
                def _c1():
                    wait(xlr1, xrr1, sem1)
                    compute(sb * (SBE // 16) + c1, c1 * 16, xlr1, xrr1)

                return cy2

            lax.fori_loop(0, (nchk + 1) // 2, pair_body, 0)
            return cy

        lax.fori_loop(0, nsb, sb_body, 0)

        def nr(row, cy):
            dinvv = 1.0 / (acc[row, pl.ds(HD, 16)] + 1e-16)
            for h in range(H):
                dv = jnp.full((16,), dinvv[h], f32)
                for j in range(C // 16):
                    col = h * C + j * 16
                    o = acc[row, pl.ds(col, 16)] * dv * pav[pl.ds(col, 16)] \
                        + pbv[pl.ds(col, 16)]
                    acc[row, pl.ds(col, 16)] = jnp.maximum(o, 0.0)
            return cy
        lax.fori_loop(0, BNB, nr, 0)

        nrows = jnp.clip(N - base, 0, BNB)
        nchw = nrows // 16

        def wr(q, cy):
            pltpu.sync_copy(acc.at[pl.ds(q * 16, 16)],
                            hout.at[pl.ds(base + q * 16, 16)])
            return cy
        lax.fori_loop(0, nchw, wr, 0)
        return carry

    lax.fori_loop(0, BPT, bin_body, 0)


def _make_layer_kernel():
    mesh = plsc.VectorSubcoreMesh(core_axis_name="c", subcore_axis_name="s")
    return pl.kernel(
        _layer_body,
        out_type=jax.ShapeDtypeStruct((NPAD, AW), f32),
        mesh=mesh,
        scratch_types=[
            pltpu.VMEM((HD,), f32),
            pltpu.VMEM((HD,), f32),
            pltpu.VMEM((HD,), f32),
            pltpu.VMEM((16, HD), f32),
            pltpu.VMEM((16, HD), f32),
            pltpu.VMEM((16, HD), f32),
            pltpu.VMEM((16, HD), f32),
            pltpu.VMEM((BNB, AW), f32),
            pltpu.VMEM((SBE,), i32),
            pltpu.VMEM((32,), i32),
            pltpu.VMEM((32,), i32),
            pltpu.SMEM((32,), i32),
            pltpu.SemaphoreType.DMA,
            pltpu.SemaphoreType.DMA,
        ],
    )


def kernel(x, edge_index, Wl0, bl0, Wr0, br0, att0, bias0, g0, be0, rm0, rv0,
           Wl1, bl1, Wr1, br1, att1, bias1, g1, be1, rm1, rv1, W1, b1, W2, b2):
    loop = jnp.arange(N, dtype=edge_index.dtype)
    srcp = jnp.concatenate([edge_index[0], loop,
                            jnp.zeros((EPT - E - N,), i32)])
    dstp = jnp.concatenate([edge_index[1], loop,
                            jnp.full((EPT - E - N,), NPAD - 1, i32)])
    epk = srcp * PK + dstp

    cnts = _make_count_kernel()(epk)
    bpack, bstarts, bends = _make_scatter_kernel()(epk, cnts)

    s0 = g0 * lax.rsqrt(rv0 + 1e-5)
    pb0 = bias0 * s0 + (be0 - rm0 * s0)
    s1 = g1 * lax.rsqrt(rv1 + 1e-5)
    pb1 = bias1 * s1 + (be1 - rm1 * s1)

    layer = _make_layer_kernel()

    xp = jnp.pad(x, ((0, NPAD - N), (0, 0)))
    xl0, xr0 = _mm2(xp, Wl0, bl0, Wr0, br0)
    h = layer(xl0, xr0, bpack, bstarts, bends, att0.reshape(HD), s0, pb0)

    hp = jnp.pad(h[:N, :HD], ((0, NPAD - N), (0, 0)))
    xl1, xr1 = _mm2(hp, Wl1, bl1, Wr1, br1)
    h = layer(xl1, xr1, bpack, bstarts, bends, att1.reshape(HD), s1, pb1)

    hp = jnp.pad(h[:N, :HD], ((0, NPAD - N), (0, 0)))
    h = _mm(hp, W1, b1, act="relu")
    h = _mm(h, W2, b2)[:N]
    return h


# speedup vs baseline: 14.2833x; 1.0349x over previous
import functools

import jax
import jax.numpy as jnp
from jax import lax
from jax.experimental import pallas as pl
from jax.experimental.pallas import tpu as pltpu
from jax.experimental.pallas import tpu_sc as plsc

i32 = jnp.int32
f32 = jnp.float32

N = 10000
E = 160000
D = 256
H = 4
C = 256
HD = H * C
OUT = 128

NW = 32
S = 5344
EPT = NW * S
BINS = 320
BSH = 5
BNB = 32
PK = 16384
NPAD = BINS * BNB
AW = HD + 16
BPT = BINS // NW
BPK = EPT + BINS * 16 + 512
SBE = 4096
BM = 512


def _mm_kernel(x_ref, w_ref, b_ref, o_ref, *, act, kuse):
    xv = x_ref[...]
    if kuse is not None:
        xv = xv[:, :kuse]
    acc = jnp.dot(xv, w_ref[...], preferred_element_type=f32)
    acc = acc + b_ref[...][None, :]
    if act == "relu":
        acc = jnp.maximum(acc, 0.0)
    o_ref[...] = acc


def _mm(x, w, b, act=None, kuse=None):
    M, K = x.shape
    KW, Nout = w.shape
    return pl.pallas_call(
        functools.partial(_mm_kernel, act=act, kuse=kuse),
        grid=(M // BM,),
        in_specs=[
            pl.BlockSpec((BM, K), lambda i: (i, 0)),
            pl.BlockSpec((KW, Nout), lambda i: (0, 0)),
            pl.BlockSpec((Nout,), lambda i: (0,)),
        ],
        out_specs=pl.BlockSpec((BM, Nout), lambda i: (i, 0)),
        out_shape=jax.ShapeDtypeStruct((M, Nout), f32),
    )(x, w, b)


def _mm2_kernel(x_ref, wl_ref, bl_ref, wr_ref, br_ref, ol_ref, or_ref, *, kuse):
    xv = x_ref[...]
    if kuse is not None:
        xv = xv[:, :kuse]
    ol_ref[...] = jnp.dot(xv, wl_ref[...], preferred_element_type=f32) + bl_ref[...][None, :]
    or_ref[...] = jnp.dot(xv, wr_ref[...], preferred_element_type=f32) + br_ref[...][None, :]


def _mm2(x, wl, bl, wr, br, kuse=None):
    M, K = x.shape
    KW = wl.shape[0]
    return pl.pallas_call(
        functools.partial(_mm2_kernel, kuse=kuse),
        grid=(M // BM,),
        in_specs=[
            pl.BlockSpec((BM, K), lambda i: (i, 0)),
            pl.BlockSpec((KW, HD), lambda i: (0, 0)),
            pl.BlockSpec((HD,), lambda i: (0,)),
            pl.BlockSpec((KW, HD), lambda i: (0, 0)),
            pl.BlockSpec((HD,), lambda i: (0,)),
        ],
        out_specs=[
            pl.BlockSpec((BM, HD), lambda i: (i, 0)),
            pl.BlockSpec((BM, HD), lambda i: (i, 0)),
        ],
        out_shape=[
            jax.ShapeDtypeStruct((M, HD), f32),
            jax.ShapeDtypeStruct((M, HD), f32),
        ],
    )(x, wl, bl, wr, br)


def _count_body(epk, cnts, spk, stage, hist, sem):
    c = lax.axis_index("c")
    s = lax.axis_index("s")
    w = c * 16 + s
    iota = lax.iota(i32, 16)
    pltpu.sync_copy(epk.at[pl.ds(pl.multiple_of(w * S, 32), S)], spk)

    for r in range(BINS):
        hist[r] = jnp.int32(0)

    def count_loop(k, cy):
        bv = (spk[pl.ds(k * 16, 16)] & (PK - 1)) >> BSH
        for q in range(16):
            b = bv[q]
            hist[b] = hist[b] + 1
        return cy

    lax.fori_loop(0, S // 16, count_loop, 0)

    def emit(g, cy):
        vv = jnp.zeros((16,), i32)
        for q in range(16):
            vv = jnp.where(iota == q, jnp.full((16,), hist[g * 16 + q], i32), vv)
        stage[pl.ds(0, 16)] = vv
        oidx = (g * 16 + iota) * NW + w
        pltpu.sync_copy(stage.at[pl.ds(0, 16)], cnts.at[oidx])
        return cy

    lax.fori_loop(0, BINS // 16, emit, 0)


def _make_count_kernel():
    mesh = plsc.VectorSubcoreMesh(core_axis_name="c", subcore_axis_name="s")
    return pl.kernel(
        _count_body,
        out_type=jax.ShapeDtypeStruct((BINS * NW,), i32),
        mesh=mesh,
        scratch_types=[
            pltpu.VMEM((S,), i32),
            pltpu.VMEM((16,), i32),
            pltpu.SMEM((BINS,), i32),
            pltpu.SemaphoreType.DMA,
        ],
    )


def _scatter_body(epk, cnts, bpack, bstarts, bends, spk, cbuf, stage, zi,
                  hist, sem):
    c = lax.axis_index("c")
    s = lax.axis_index("s")
    w = c * 16 + s
    iota = lax.iota(i32, 16)
    pltpu.sync_copy(epk.at[pl.ds(pl.multiple_of(w * S, 32), S)], spk)
    pltpu.sync_copy(cnts, cbuf)

    def scan(b, gstart):
        v0 = cbuf[pl.ds(pl.multiple_of(b * NW, 32), 16)]
        v1 = cbuf[pl.ds(pl.multiple_of(b * NW, 32) + 16, 16)]
        tot = jnp.int32(0)
        woff = jnp.int32(0)
        for q in range(16):
            cq = v0[q]
            woff = woff + jnp.where(jnp.int32(q) < w, cq, 0)
            tot = tot + cq
        for q in range(16):
            cq = v1[q]
            woff = woff + jnp.where(jnp.int32(16 + q) < w, cq, 0)
            tot = tot + cq
        hist[b] = gstart + woff
        hist[BINS + b] = gstart + tot
        return gstart + ((tot + 15) >> 4 << 4)

    lax.fori_loop(0, BINS, scan, jnp.int32(0))

    @pl.when(w == 0)
    def _emit_bounds():
        def emit(g, cy):
            sv = jnp.zeros((16,), i32)
            evv = jnp.zeros((16,), i32)
            for q in range(16):
                st_q = hist[g * 16 + q]
                en_q = hist[BINS + g * 16 + q]
                sv = jnp.where(iota == q, jnp.full((16,), st_q, i32), sv)
                evv = jnp.where(iota == q, jnp.full((16,), en_q, i32), evv)
            stage[pl.ds(0, 16)] = sv
            stage[pl.ds(16, 16)] = evv
            pltpu.sync_copy(stage.at[pl.ds(0, 16)],
                            bstarts.at[pl.ds(pl.multiple_of(g * 16, 16), 16)])
            pltpu.sync_copy(stage.at[pl.ds(16, 16)],
                            bends.at[pl.ds(pl.multiple_of(g * 16, 16), 16)])
            return cy
        lax.fori_loop(0, BINS // 16, emit, 0)

    def scat_body(k, cy):
        bv = (spk[pl.ds(k * 16, 16)] & (PK - 1)) >> BSH
        posv = jnp.zeros((16,), i32)
        for q in range(16):
            b = bv[q]
            p = hist[b]
            hist[b] = p + 1
            posv = jnp.where(iota == q, jnp.full((16,), p, i32), posv)
        pltpu.sync_copy(spk.at[pl.ds(k * 16, 16)], bpack.at[posv])
        return cy

    lax.fori_loop(0, S // 16, scat_body, 0)


def _make_scatter_kernel():
    mesh = plsc.VectorSubcoreMesh(core_axis_name="c", subcore_axis_name="s")
    return pl.kernel(
        _scatter_body,
        out_type=(
            jax.ShapeDtypeStruct((BPK,), i32),
            jax.ShapeDtypeStruct((BINS,), i32),
            jax.ShapeDtypeStruct((BINS,), i32),
        ),
        mesh=mesh,
        scratch_types=[
            pltpu.VMEM((S,), i32),
            pltpu.VMEM((BINS * NW,), i32),
            pltpu.VMEM((32,), i32),
            pltpu.VMEM((16,), i32),
            pltpu.SMEM((2 * BINS,), i32),
            pltpu.SemaphoreType.DMA,
        ],
    )


def _layer_body(xl, xr, bpack, bstarts, bends, atth, pah, pbh, hout,
                attv, pav, pbv, xlr0, xrr0, xlr1, xrr1, acc, ebuf, rvb,
                stv, smb, sem0, sem1):
    c = lax.axis_index("c")
    s = lax.axis_index("s")
    tid = c * 16 + s
    iota = lax.iota(i32, 16)
    zf = jnp.zeros((16,), f32)

    pltpu.sync_copy(atth, attv)
    pltpu.sync_copy(pah, pav)
    pltpu.sync_copy(pbh, pbv)

    b0 = tid * BPT
    off8 = pl.multiple_of((b0 >> 3) << 3, 8)
    shift = b0 - off8
    pltpu.sync_copy(bstarts.at[pl.ds(off8, 16)], stv.at[pl.ds(0, 16)])
    pltpu.sync_copy(bends.at[pl.ds(off8, 16)], stv.at[pl.ds(16, 16)])
    sv = stv[pl.ds(0, 16)]
    evv = stv[pl.ds(16, 16)]
    for q in range(16):
        smb[q] = sv[q]
        smb[16 + q] = evv[q]

    def issue(pos16, bxl, bxr, semx):
        pkv = ebuf[pl.ds(pos16, 16)]
        srcv = jnp.clip(pkv >> 14, 0, NPAD - 1)
        dstv = jnp.minimum(pkv & (PK - 1), NPAD - 1)
        pltpu.async_copy(xl.at[srcv], bxl, semx)
        pltpu.async_copy(xr.at[dstv], bxr, semx)

    def wait(bxl, bxr, semx):
        pltpu.make_async_copy(xl.at[iota], bxl, semx).wait()
        pltpu.make_async_copy(xr.at[iota], bxr, semx).wait()

    def bin_body(bi, carry):
        bn = tid * BPT + bi
        base = bn * BNB
        st = smb[shift + bi]
        en = smb[16 + shift + bi]
        cnt = en - st

        def zb(row, cy):
            for g in range(AW // 16):
                acc[row, pl.ds(g * 16, 16)] = zf
            return cy
        lax.fori_loop(0, BNB, zb, 0)

        def compute(cidx, local16, bxl, bxr):
            pkv = ebuf[pl.ds(local16, 16)]
            dstv = jnp.minimum(pkv & (PK - 1), NPAD - 1)
            rvb[pl.ds(0, 16)] = dstv - base
            nv = jnp.clip(cnt - cidx * 16, 0, 16)

            def tree_sum(part):
                tree = [part[q] for q in range(16)]
                while len(tree) > 1:
                    tree = [tree[2 * q] + tree[2 * q + 1]
                            for q in range(len(tree) // 2)]
                return tree[0]

            def ptail_of(pvs):
                return jnp.where(iota == 0, pvs[0],
                        jnp.where(iota == 1, pvs[1],
                         jnp.where(iota == 2, pvs[2],
                          jnp.where(iota == 3, pvs[3], zf))))

            def pair_edges(m, cy3):
                e0 = 2 * m
                e1 = 2 * m + 1
                row0 = rvb[pl.ds(e0, 16)][0]
                row1 = rvb[pl.ds(e1, 16)][0]
                pvs0 = []
                pvs1 = []
                for h in range(H):
                    p0 = zf
                    p1 = zf
                    xs0 = []
                    xs1 = []
                    for j in range(C // 16):
                        col = h * C + j * 16
                        av = attv[pl.ds(col, 16)]
                        x0 = bxl[e0, pl.ds(col, 16)]
                        x1 = bxl[e1, pl.ds(col, 16)]
                        xs0.append(x0)
                        xs1.append(x1)
                        t0 = x0 + bxr[e0, pl.ds(col, 16)]
                        p0 = p0 + jnp.maximum(t0, 0.2 * t0) * av
                        t1 = x1 + bxr[e1, pl.ds(col, 16)]
                        p1 = p1 + jnp.maximum(t1, 0.2 * t1) * av
                    pv0 = jnp.exp(jnp.full((16,), tree_sum(p0), f32))
                    pv1 = jnp.exp(jnp.full((16,), tree_sum(p1), f32))
                    pvs0.append(pv0)
                    pvs1.append(pv1)
                    for j in range(C // 16):
                        col = h * C + j * 16
                        plsc.addupdate(acc.at[row0, pl.ds(col, 16)],
                                       pv0 * xs0[j])
                        plsc.addupdate(acc.at[row1, pl.ds(col, 16)],
                                       pv1 * xs1[j])
                plsc.addupdate(acc.at[row0, pl.ds(HD, 16)], ptail_of(pvs0))
                plsc.addupdate(acc.at[row1, pl.ds(HD, 16)], ptail_of(pvs1))
                return cy3

            def edge_body(e, cy3):
                row = rvb[pl.ds(e, 16)][0]
                pvs = []
                for h in range(H):
                    part = zf
                    for j in range(C // 16):
                        col = h * C + j * 16
                        t = bxl[e, pl.ds(col, 16)] + bxr[e, pl.ds(col, 16)]
                        part = part + jnp.maximum(t, 0.2 * t) * attv[pl.ds(col, 16)]
                    pv = jnp.exp(jnp.full((16,), tree_sum(part), f32))
                    pvs.append(pv)
                    for j in range(C // 16):
                        col = h * C + j * 16
                        plsc.addupdate(acc.at[row, pl.ds(col, 16)],
                                       pv * bxl[e, pl.ds(col, 16)])
                plsc.addupdate(acc.at[row, pl.ds(HD, 16)], ptail_of(pvs))
                return cy3

            npair = nv // 2
            lax.fori_loop(0, npair, pair_edges, 0)
            lax.fori_loop(npair * 2, nv, edge_body, 0)

        nsb = (cnt + SBE - 1) // SBE

        def sb_body(sb, cy):
            sbase = pl.multiple_of(st + sb * SBE, 16)
            rem = cnt - sb * SBE
            nblk = jnp.minimum((rem + 255) // 256, SBE // 256)

            def ld(t2, cy2):
                o = pl.multiple_of(t2 * 256, 256)
                pltpu.sync_copy(bpack.at[pl.ds(sbase + o, 256)],
                                ebuf.at[pl.ds(o, 256)])
                return cy2
            lax.fori_loop(0, nblk, ld, 0)

            nchk = jnp.clip((rem + 15) // 16, 0, SBE // 16)

            @pl.when(nchk > 0)
            def _pro():
                issue(0, xlr0, xrr0, sem0)

            def pair_body(m, cy2):
                c0 = 2 * m
                c1 = 2 * m + 1
                cond1 = c1 < nchk
                cond2 = c1 + 1 < nchk
                wait(xlr0, xrr0, sem0)

                @pl.when(cond1)
                def _i1():
                    issue(c1 * 16, xlr1, xrr1, sem1)

                compute(sb * (SBE // 16) + c0, c0 * 16, xlr0, xrr0)

                @pl.when(cond2)
                def _i2():
                    issue((c1 + 1) * 16, xlr0, xrr0, sem0)

                @pl.when(cond1)
                def _c1():
                    wait(xlr1, xrr1, sem1)
                    compute(sb * (SBE // 16) + c1, c1 * 16, xlr1, xrr1)

                return cy2

            lax.fori_loop(0, (nchk + 1) // 2, pair_body, 0)
            return cy

        lax.fori_loop(0, nsb, sb_body, 0)

        def nr(row, cy):
            dinvv = 1.0 / (acc[row, pl.ds(HD, 16)] + 1e-16)
            for h in range(H):
                dv = jnp.full((16,), dinvv[h], f32)
                for j in range(C // 16):
                    col = h * C + j * 16
                    o = acc[row, pl.ds(col, 16)] * dv * pav[pl.ds(col, 16)] \
                        + pbv[pl.ds(col, 16)]
                    acc[row, pl.ds(col, 16)] = jnp.maximum(o, 0.0)
            return cy
        lax.fori_loop(0, BNB, nr, 0)

        nrows = jnp.clip(N - base, 0, BNB)
        nchw = nrows // 16

        def wr(q, cy):
            pltpu.sync_copy(acc.at[pl.ds(q * 16, 16)],
                            hout.at[pl.ds(base + q * 16, 16)])
            return cy
        lax.fori_loop(0, nchw, wr, 0)
        return carry

    lax.fori_loop(0, BPT, bin_body, 0)


def _make_layer_kernel():
    mesh = plsc.VectorSubcoreMesh(core_axis_name="c", subcore_axis_name="s")
    return pl.kernel(
        _layer_body,
        out_type=jax.ShapeDtypeStruct((NPAD, AW), f32),
        mesh=mesh,
        scratch_types=[
            pltpu.VMEM((HD,), f32),
            pltpu.VMEM((HD,), f32),
            pltpu.VMEM((HD,), f32),
            pltpu.VMEM((16, HD), f32),
            pltpu.VMEM((16, HD), f32),
            pltpu.VMEM((16, HD), f32),
            pltpu.VMEM((16, HD), f32),
            pltpu.VMEM((BNB, AW), f32),
            pltpu.VMEM((SBE,), i32),
            pltpu.VMEM((32,), i32),
            pltpu.VMEM((32,), i32),
            pltpu.SMEM((32,), i32),
            pltpu.SemaphoreType.DMA,
            pltpu.SemaphoreType.DMA,
        ],
    )


def kernel(x, edge_index, Wl0, bl0, Wr0, br0, att0, bias0, g0, be0, rm0, rv0,
           Wl1, bl1, Wr1, br1, att1, bias1, g1, be1, rm1, rv1, W1, b1, W2, b2):
    loop = jnp.arange(N, dtype=edge_index.dtype)
    srcp = jnp.concatenate([edge_index[0], loop,
                            jnp.zeros((EPT - E - N,), i32)])
    dstp = jnp.concatenate([edge_index[1], loop,
                            jnp.full((EPT - E - N,), NPAD - 1, i32)])
    epk = srcp * PK + dstp

    cnts = _make_count_kernel()(epk)
    bpack, bstarts, bends = _make_scatter_kernel()(epk, cnts)

    s0 = g0 * lax.rsqrt(rv0 + 1e-5)
    pb0 = bias0 * s0 + (be0 - rm0 * s0)
    s1 = g1 * lax.rsqrt(rv1 + 1e-5)
    pb1 = bias1 * s1 + (be1 - rm1 * s1)

    layer = _make_layer_kernel()

    xp = jnp.pad(x, ((0, NPAD - N), (0, 0)))
    xl0, xr0 = _mm2(xp, Wl0, bl0, Wr0, br0)
    h = layer(xl0, xr0, bpack, bstarts, bends, att0.reshape(HD), s0, pb0)

    xl1, xr1 = _mm2(h, Wl1, bl1, Wr1, br1, kuse=HD)
    h = layer(xl1, xr1, bpack, bstarts, bends, att1.reshape(HD), s1, pb1)

    h = _mm(h, W1, b1, act="relu", kuse=HD)
    h = _mm(h, W2, b2)[:N]
    return h
